# baseline (speedup 1.0000x reference)
import functools

import jax
import jax.numpy as jnp
from jax import lax
from jax.experimental import pallas as pl
from jax.experimental.pallas import tpu as pltpu
from jax.experimental.pallas import tpu_sc as plsc

BT = 8
VH_RAW = 6890
VO = 2048
TH = 128
NT = 54
VH = TH * NT
THR = 0.2
EPS = 1e-8
BIGC = 1e9

_NSLICE = 4
_HS = VH // _NSLICE
_OS = VO // _NSLICE


def _dist_body(a_ref, bt_ref, rmin_ref, rarg_ref, cmin_ref, carg_ref):
    i = pl.program_id(1)
    a = a_ref[0]
    bt = bt_ref[0]
    d2 = lax.dot_general(a, bt, (((1,), (0,)), ((), ())),
                         preferred_element_type=jnp.float32)
    exp23 = jnp.int32(0x4B000000)
    sent = jnp.float32(16777216.0)
    m23 = jnp.int32(0x7FFFFF)
    rmin = jnp.min(d2, axis=1)
    col_iota = lax.bitcast_convert_type(
        lax.broadcasted_iota(jnp.int32, (1, VO), 1) | exp23, jnp.float32)
    rarg_f = jnp.min(
        jnp.where(d2 == rmin[:, None], jnp.broadcast_to(col_iota, (TH, VO)),
                  sent), axis=1)
    rmin_ref[0, 0, :] = rmin
    rarg_ref[0, 0, :] = lax.bitcast_convert_type(rarg_f, jnp.int32) & m23
    cmin_t = jnp.min(d2, axis=0)
    row_iota = lax.bitcast_convert_type(
        (lax.broadcasted_iota(jnp.int32, (TH, 1), 0) + i * TH) | exp23,
        jnp.float32)
    carg_f = jnp.min(
        jnp.where(d2 == cmin_t[None, :], jnp.broadcast_to(row_iota, (TH, VO)),
                  sent), axis=0)
    carg_t = lax.bitcast_convert_type(carg_f, jnp.int32) & m23

    @pl.when(i == 0)
    def _init():
        cmin_ref[0, 0, :] = cmin_t
        carg_ref[0, 0, :] = carg_t

    @pl.when(i != 0)
    def _acc():
        prev = cmin_ref[0, 0, :]
        parg = carg_ref[0, 0, :]
        better = cmin_t < prev
        cmin_ref[0, 0, :] = jnp.where(better, cmin_t, prev)
        carg_ref[0, 0, :] = jnp.where(better, carg_t, parg)


def _loss_body(d2h_ref, d2o_ref, diffh_ref, diffo_ref, out_ref):
    basis_h = jnp.sqrt(jnp.maximum(d2h_ref[...], 1e-12))
    basis_o = jnp.sqrt(jnp.maximum(d2o_ref[...], 1e-12))
    mask_h = basis_h < THR
    mask_o = basis_o < THR
    sel_h = jnp.sum(mask_h.astype(jnp.int32), axis=1)
    sel_o = jnp.sum(mask_o.astype(jnp.int32), axis=1)

    def kth(sel):
        return jnp.maximum(
            1,
            jnp.round(jnp.float32(0.2) * sel.astype(jnp.float32)).astype(
                jnp.int32))

    k_h = kth(sel_h)
    k_o = kth(sel_o)
    bits_h = lax.bitcast_convert_type(basis_h, jnp.int32)
    bits_o = lax.bitcast_convert_type(basis_o, jnp.int32)

    def step(_, carry):
        lo_h, hi_h, lo_o, hi_o = carry

        def halve(lo, hi, mask, bits, k):
            mid = lo + lax.div(hi - lo, 2)
            cnt = jnp.sum(jnp.where(mask & (bits <= mid[:, None]), 1, 0),
                          axis=1)
            ge = cnt >= k
            return jnp.where(ge, lo, mid + 1), jnp.where(ge, mid, hi)

        lo_h, hi_h = halve(lo_h, hi_h, mask_h, bits_h, k_h)
        lo_o, hi_o = halve(lo_o, hi_o, mask_o, bits_o, k_o)
        return lo_h, hi_h, lo_o, hi_o

    z = jnp.zeros((8,), jnp.int32)
    inf = jnp.full((8,), 0x7F800000, jnp.int32)
    lo_h, _, lo_o, _ = lax.fori_loop(0, 31, step, (z, inf, z, inf))

    def branch(lo, sel, mask, basis, diff_ref):
        t = lax.bitcast_convert_type(lo, jnp.float32)
        t = jnp.where(sel > 0, t, jnp.float32(1.0))
        w = jnp.maximum((t[:, None] - basis) / (t[:, None] + EPS), 0.0)
        w2 = w * w
        w4 = jnp.where(mask, w2 * w2, 0.0)
        return jnp.sum(w4 * diff_ref[...]) / (jnp.sum(w4) + EPS)

    l_h = branch(lo_h, sel_h, mask_h, basis_h, diffh_ref)
    l_o = branch(lo_o, sel_o, mask_o, basis_o, diffo_ref)
    out_ref[...] = jnp.broadcast_to(l_h + l_o, (1, 1))


@functools.lru_cache(maxsize=1)
def _build_gather():
    mesh = plsc.VectorSubcoreMesh(core_axis_name="c", subcore_axis_name="s")
    return functools.partial(
        pl.kernel,
        mesh=mesh,
        out_type=[jax.ShapeDtypeStruct((BT * VH,), jnp.float32),
                  jax.ShapeDtypeStruct((BT * VO,), jnp.float32)],
        scratch_types=_GATHER_SCRATCH,
        compiler_params=pltpu.CompilerParams(needs_layout_passes=False),
    )(_gather_body)


_GATHER_SCRATCH = [
        pltpu.VMEM((VH * 3,), jnp.float32), pltpu.VMEM((VH * 3,), jnp.float32),
        pltpu.VMEM((VO * 3,), jnp.float32), pltpu.VMEM((VO * 3,), jnp.float32),
        pltpu.VMEM((_HS,), jnp.int32), pltpu.VMEM((_OS,), jnp.int32),
        pltpu.VMEM((_HS,), jnp.float32), pltpu.VMEM((_OS,), jnp.float32),
]


def _gather_body(gh, ph, go, po, idxo, idxh, diffh_out, diffo_out,
                 t_gh, t_ph, t_go, t_po, t_idxo, t_idxh, t_dh, t_do):
    wid = lax.axis_index("s") * 2 + lax.axis_index("c")
    b = wid // _NSLICE
    s = wid % _NSLICE
    hbase = b * VH
    obase = b * VO
    pltpu.sync_copy(gh.at[pl.ds(hbase * 3, VH * 3)], t_gh)
    pltpu.sync_copy(ph.at[pl.ds(hbase * 3, VH * 3)], t_ph)
    pltpu.sync_copy(go.at[pl.ds(obase * 3, VO * 3)], t_go)
    pltpu.sync_copy(po.at[pl.ds(obase * 3, VO * 3)], t_po)
    hoff = s * _HS
    ooff = s * _OS
    pltpu.sync_copy(idxo.at[pl.ds(hbase + hoff, _HS)], t_idxo)
    pltpu.sync_copy(idxh.at[pl.ds(obase + ooff, _OS)], t_idxh)
    lane3 = lax.iota(jnp.int32, 16) * 3

    def diff16_rel(nn_g_tab, nn_p_tab, own_g_tab, own_p_tab, own_base, idx):
        i3 = idx * 3
        own3 = lane3 + own_base * 3
        d = None
        for c in range(3):
            g_nn = plsc.load_gather(nn_g_tab, [i3 + c])
            p_nn = plsc.load_gather(nn_p_tab, [i3 + c])
            g_own = plsc.load_gather(own_g_tab, [own3 + c])
            p_own = plsc.load_gather(own_p_tab, [own3 + c])
            dc = jnp.abs((p_nn - p_own) - (g_nn - g_own))
            d = dc if d is None else d + dc
        return d

    def hstep(c, carry):
        base = c * 16
        idx = t_idxo[pl.ds(base, 16)]
        t_dh[pl.ds(base, 16)] = diff16_rel(t_go, t_po, t_gh, t_ph,
                                           hoff + base, idx)
        return carry

    lax.fori_loop(0, _HS // 16, hstep, 0)

    def ostep(c, carry):
        base = c * 16
        idx = t_idxh[pl.ds(base, 16)]
        t_do[pl.ds(base, 16)] = diff16_rel(t_gh, t_ph, t_go, t_po,
                                           ooff + base, idx)
        return carry

    lax.fori_loop(0, _OS // 16, ostep, 0)

    pltpu.sync_copy(t_dh, diffh_out.at[pl.ds(hbase + hoff, _HS)])
    pltpu.sync_copy(t_do, diffo_out.at[pl.ds(obase + ooff, _OS)])


def _dist_call(A, Bt):
    return pl.pallas_call(
        _dist_body,
        grid=(BT, NT),
        in_specs=[
            pl.BlockSpec((1, TH, 8), lambda b, i: (b, i, 0)),
            pl.BlockSpec((1, 8, VO), lambda b, i: (b, 0, 0)),
        ],
        out_specs=[
            pl.BlockSpec((1, 1, TH), lambda b, i: (b * NT + i, 0, 0)),
            pl.BlockSpec((1, 1, TH), lambda b, i: (b * NT + i, 0, 0)),
            pl.BlockSpec((1, 1, VO), lambda b, i: (b, 0, 0)),
            pl.BlockSpec((1, 1, VO), lambda b, i: (b, 0, 0)),
        ],
        out_shape=[
            jax.ShapeDtypeStruct((BT * NT, 1, TH), jnp.float32),
            jax.ShapeDtypeStruct((BT * NT, 1, TH), jnp.int32),
            jax.ShapeDtypeStruct((BT, 1, VO), jnp.float32),
            jax.ShapeDtypeStruct((BT, 1, VO), jnp.int32),
        ],
        compiler_params=pltpu.CompilerParams(
            dimension_semantics=("arbitrary", "arbitrary")),
    )(A, Bt)


def _loss_call(d2h, d2o, diffh, diffo):
    return pl.pallas_call(
        _loss_body,
        out_shape=jax.ShapeDtypeStruct((1, 1), jnp.float32),
    )(d2h, d2o, diffh, diffo)


def _run_gather(gh, ph, go, po, idxo, idxh):
    return _build_gather()(
        gh.reshape(-1), ph.reshape(-1), go.reshape(-1), po.reshape(-1),
        idxo.reshape(-1), idxh.reshape(-1))


def kernel(pred_h_verts, pred_o_verts, gt_h_verts, gt_o_verts):
    pad = VH - VH_RAW
    gh = jnp.pad(gt_h_verts, ((0, 0), (0, pad), (0, 0)), constant_values=BIGC)
    ph = jnp.pad(pred_h_verts, ((0, 0), (0, pad), (0, 0)))
    go = gt_o_verts
    po = pred_o_verts
    a2 = jnp.sum(gh * gh, axis=-1, keepdims=True)
    b2 = jnp.sum(go * go, axis=-1, keepdims=True)
    A = jnp.concatenate([gh, a2, jnp.ones_like(a2), jnp.zeros_like(gh)], axis=-1)
    Bm = jnp.concatenate(
        [-2.0 * go, jnp.ones_like(b2), b2, jnp.zeros_like(go)], axis=-1)
    Bt = jnp.transpose(Bm, (0, 2, 1))

    rmin3, rarg3, cmin3, carg3 = _dist_call(A, Bt)
    rmin2 = rmin3.reshape(BT, VH)
    idxo = rarg3.reshape(BT, VH)
    cmin2 = cmin3.reshape(BT, VO)
    idxh = carg3.reshape(BT, VO)

    diffh_flat, diffo_flat = _run_gather(gh, ph, go, po, idxo, idxh)
    diffh = diffh_flat.reshape(BT, VH)
    diffo = diffo_flat.reshape(BT, VO)

    out = _loss_call(rmin2, cmin2, diffh, diffo)
    return out[0, 0]


# speedup vs baseline: 2.1900x; 2.1900x over previous
import functools

import jax
import jax.numpy as jnp
from jax import lax
from jax.experimental import pallas as pl
from jax.experimental.pallas import tpu as pltpu
from jax.experimental.pallas import tpu_sc as plsc

BT = 8
VH_RAW = 6890
VO = 2048
TH = 128
NT = 54
VH = TH * NT
THR = 0.2
EPS = 1e-8
BIGC = 1e9

_NSLICE = 4
_HS = VH // 3
_OS = VO


def _dist_body(a_ref, bt_ref, rmin_ref, rarg_ref, cmin_ref, carg_ref):
    i = pl.program_id(0)
    exp23 = jnp.int32(0x4B000000)
    sent = jnp.float32(16777216.0)
    m23 = jnp.int32(0x7FFFFF)
    col_iota = lax.bitcast_convert_type(
        lax.broadcasted_iota(jnp.int32, (1, VO), 1) | exp23, jnp.float32)
    row_iota = lax.bitcast_convert_type(
        (lax.broadcasted_iota(jnp.int32, (TH, 1), 0) + i * TH) | exp23,
        jnp.float32)
    rmins, rargs, cmins, cargs = [], [], [], []
    for bb in range(BT):
        a = a_ref[bb]
        bt = bt_ref[bb]
        d2 = lax.dot_general(a, bt, (((0,), (0,)), ((), ())),
                             preferred_element_type=jnp.float32)
        rmin = jnp.min(d2, axis=1)
        rarg_f = jnp.min(
            jnp.where(d2 == rmin[:, None],
                      jnp.broadcast_to(col_iota, (TH, VO)), sent), axis=1)
        rmins.append(rmin)
        rargs.append(lax.bitcast_convert_type(rarg_f, jnp.int32) & m23)
        cmin_t = jnp.min(d2, axis=0)
        carg_f = jnp.min(
            jnp.where(d2 == cmin_t[None, :],
                      jnp.broadcast_to(row_iota, (TH, VO)), sent), axis=0)
        cmins.append(cmin_t)
        cargs.append(lax.bitcast_convert_type(carg_f, jnp.int32) & m23)

    rmin_ref[:, 0, :] = jnp.stack(rmins)
    rarg_ref[:, 0, :] = jnp.stack(rargs)
    cmin_all = jnp.stack(cmins)
    carg_all = jnp.stack(cargs)

    @pl.when(i == 0)
    def _init():
        cmin_ref[:, 0, :] = cmin_all
        carg_ref[:, 0, :] = carg_all

    @pl.when(i != 0)
    def _acc():
        prev = cmin_ref[:, 0, :]
        parg = carg_ref[:, 0, :]
        better = cmin_all < prev
        cmin_ref[:, 0, :] = jnp.where(better, cmin_all, prev)
        carg_ref[:, 0, :] = jnp.where(better, carg_all, parg)


def _loss_body(d2h_ref, d2o_ref, diffh_ref, diffo_ref, out_ref):
    basis_h = jnp.sqrt(jnp.maximum(d2h_ref[...], 1e-12))
    basis_o = jnp.sqrt(jnp.maximum(d2o_ref[...], 1e-12))
    mask_h = basis_h < THR
    mask_o = basis_o < THR
    sel_h = jnp.sum(mask_h.astype(jnp.int32), axis=-1)
    sel_o = jnp.sum(mask_o.astype(jnp.int32), axis=-1)

    def kth(sel):
        return jnp.maximum(
            1,
            jnp.round(jnp.float32(0.2) * sel.astype(jnp.float32)).astype(
                jnp.int32))

    k_h = kth(sel_h)
    k_o = kth(sel_o)
    bits_h = lax.bitcast_convert_type(basis_h, jnp.int32)
    bits_o = lax.bitcast_convert_type(basis_o, jnp.int32)

    def step(_, carry):
        lo_h, hi_h, lo_o, hi_o = carry

        def halve(lo, hi, mask, bits, k):
            mid = lo + lax.div(hi - lo, 2)
            cnt = jnp.sum(jnp.where(mask & (bits <= mid[..., None]), 1, 0),
                          axis=-1)
            ge = cnt >= k
            return jnp.where(ge, lo, mid + 1), jnp.where(ge, mid, hi)

        lo_h, hi_h = halve(lo_h, hi_h, mask_h, bits_h, k_h)
        lo_o, hi_o = halve(lo_o, hi_o, mask_o, bits_o, k_o)
        return lo_h, hi_h, lo_o, hi_o

    z = jnp.zeros((8, 1), jnp.int32)
    inf = jnp.full((8, 1), 0x7F800000, jnp.int32)
    lo_h, _, lo_o, _ = lax.fori_loop(0, 31, step, (z, inf, z, inf))

    def branch(lo, sel, mask, basis, diff_ref):
        t = lax.bitcast_convert_type(lo, jnp.float32)
        t = jnp.where(sel > 0, t, jnp.float32(1.0))[..., None]
        w = jnp.maximum((t - basis) / (t + EPS), 0.0)
        w2 = w * w
        w4 = jnp.where(mask, w2 * w2, 0.0)
        return jnp.sum(w4 * diff_ref[...]) / (jnp.sum(w4) + EPS)

    l_h = branch(lo_h, sel_h, mask_h, basis_h, diffh_ref)
    l_o = branch(lo_o, sel_o, mask_o, basis_o, diffo_ref)
    out_ref[...] = jnp.broadcast_to(l_h + l_o, (1, 1))


@functools.lru_cache(maxsize=1)
def _build_gather():
    mesh = plsc.VectorSubcoreMesh(core_axis_name="c", subcore_axis_name="s")
    return functools.partial(
        pl.kernel,
        mesh=mesh,
        out_type=[jax.ShapeDtypeStruct((BT, 1, VH), jnp.float32),
                  jax.ShapeDtypeStruct((BT, 1, VO), jnp.float32)],
        scratch_types=_GATHER_SCRATCH,
        compiler_params=pltpu.CompilerParams(needs_layout_passes=False),
    )(_gather_body)


_GATHER_SCRATCH = [
        pltpu.VMEM((VH,), jnp.float32), pltpu.VMEM((VH,), jnp.float32),
        pltpu.VMEM((VH,), jnp.float32), pltpu.VMEM((VH,), jnp.float32),
        pltpu.VMEM((VH,), jnp.float32), pltpu.VMEM((VH,), jnp.float32),
        pltpu.VMEM((VO,), jnp.float32), pltpu.VMEM((VO,), jnp.float32),
        pltpu.VMEM((VO,), jnp.float32), pltpu.VMEM((VO,), jnp.float32),
        pltpu.VMEM((VO,), jnp.float32), pltpu.VMEM((VO,), jnp.float32),
        pltpu.VMEM((_HS,), jnp.int32), pltpu.VMEM((_OS,), jnp.int32),
        pltpu.VMEM((_HS,), jnp.float32), pltpu.VMEM((_OS,), jnp.float32),
]


def _gather_body(allh, allo, idxo, idxh, diffh_out, diffo_out,
                 t_ghx, t_ghy, t_ghz, t_phx, t_phy, t_phz,
                 t_gox, t_goy, t_goz, t_pox, t_poy, t_poz,
                 t_idxo, t_idxh, t_dh, t_do):
    wid = lax.axis_index("s") * 2 + lax.axis_index("c")
    b = wid // _NSLICE
    s = wid % _NSLICE
    pltpu.sync_copy(allh.at[0, b, 0, 0, :], t_ghx)
    pltpu.sync_copy(allh.at[0, b, 1, 0, :], t_ghy)
    pltpu.sync_copy(allh.at[0, b, 2, 0, :], t_ghz)
    pltpu.sync_copy(allh.at[1, b, 0, 0, :], t_phx)
    pltpu.sync_copy(allh.at[1, b, 1, 0, :], t_phy)
    pltpu.sync_copy(allh.at[1, b, 2, 0, :], t_phz)
    pltpu.sync_copy(allo.at[0, b, 0, 0, :], t_gox)
    pltpu.sync_copy(allo.at[0, b, 1, 0, :], t_goy)
    pltpu.sync_copy(allo.at[0, b, 2, 0, :], t_goz)
    pltpu.sync_copy(allo.at[1, b, 0, 0, :], t_pox)
    pltpu.sync_copy(allo.at[1, b, 1, 0, :], t_poy)
    pltpu.sync_copy(allo.at[1, b, 2, 0, :], t_poz)

    @pl.when(s < 3)
    def _hside():
        hoff = s * _HS
        pltpu.sync_copy(idxo.at[b, 0, pl.ds(hoff, _HS)], t_idxo)

        def hstep(c, carry):
            base = c * 16
            idx = t_idxo[pl.ds(base, 16)]
            gx = plsc.load_gather(t_gox, [idx])
            gy = plsc.load_gather(t_goy, [idx])
            gz = plsc.load_gather(t_goz, [idx])
            px = plsc.load_gather(t_pox, [idx])
            py = plsc.load_gather(t_poy, [idx])
            pz = plsc.load_gather(t_poz, [idx])
            o = hoff + base
            dx = (px - t_phx[pl.ds(o, 16)]) - (gx - t_ghx[pl.ds(o, 16)])
            dy = (py - t_phy[pl.ds(o, 16)]) - (gy - t_ghy[pl.ds(o, 16)])
            dz = (pz - t_phz[pl.ds(o, 16)]) - (gz - t_ghz[pl.ds(o, 16)])
            t_dh[pl.ds(base, 16)] = jnp.abs(dx) + jnp.abs(dy) + jnp.abs(dz)
            return carry

        lax.fori_loop(0, _HS // 16, hstep, 0)
        pltpu.sync_copy(t_dh, diffh_out.at[b, 0, pl.ds(hoff, _HS)])

    @pl.when(s == 3)
    def _oside():
        pltpu.sync_copy(idxh.at[b, 0, :], t_idxh)

        def ostep(c, carry):
            base = c * 16
            idx = t_idxh[pl.ds(base, 16)]
            gx = plsc.load_gather(t_ghx, [idx])
            gy = plsc.load_gather(t_ghy, [idx])
            gz = plsc.load_gather(t_ghz, [idx])
            px = plsc.load_gather(t_phx, [idx])
            py = plsc.load_gather(t_phy, [idx])
            pz = plsc.load_gather(t_phz, [idx])
            dx = (px - t_pox[pl.ds(base, 16)]) - (gx - t_gox[pl.ds(base, 16)])
            dy = (py - t_poy[pl.ds(base, 16)]) - (gy - t_goy[pl.ds(base, 16)])
            dz = (pz - t_poz[pl.ds(base, 16)]) - (gz - t_goz[pl.ds(base, 16)])
            t_do[pl.ds(base, 16)] = jnp.abs(dx) + jnp.abs(dy) + jnp.abs(dz)
            return carry

        lax.fori_loop(0, _OS // 16, ostep, 0)
        pltpu.sync_copy(t_do, diffo_out.at[b, 0, :])


def _dist_call(ght, bt):
    return pl.pallas_call(
        _dist_body,
        grid=(NT,),
        in_specs=[
            pl.BlockSpec((BT, 8, TH), lambda i: (0, 0, i)),
            pl.BlockSpec((BT, 8, VO), lambda i: (0, 0, 0)),
        ],
        out_specs=[
            pl.BlockSpec((BT, 1, TH), lambda i: (0, 0, i)),
            pl.BlockSpec((BT, 1, TH), lambda i: (0, 0, i)),
            pl.BlockSpec((BT, 1, VO), lambda i: (0, 0, 0)),
            pl.BlockSpec((BT, 1, VO), lambda i: (0, 0, 0)),
        ],
        out_shape=[
            jax.ShapeDtypeStruct((BT, 1, VH), jnp.float32),
            jax.ShapeDtypeStruct((BT, 1, VH), jnp.int32),
            jax.ShapeDtypeStruct((BT, 1, VO), jnp.float32),
            jax.ShapeDtypeStruct((BT, 1, VO), jnp.int32),
        ],
        compiler_params=pltpu.CompilerParams(
            dimension_semantics=("arbitrary",),
            fuse_transposed_lhs_in_matmul=True),
    )(ght, bt)


def _loss_call(d2h, d2o, diffh, diffo):
    return pl.pallas_call(
        _loss_body,
        out_shape=jax.ShapeDtypeStruct((1, 1), jnp.float32),
    )(d2h, d2o, diffh, diffo)


def _run_gather(allh, allo, idxo, idxh):
    return _build_gather()(allh, allo, idxo, idxh)


def kernel(pred_h_verts, pred_o_verts, gt_h_verts, gt_o_verts):
    pad = VH - VH_RAW
    gh = jnp.pad(gt_h_verts, ((0, 0), (0, pad), (0, 0)), constant_values=BIGC)
    ph = jnp.pad(pred_h_verts, ((0, 0), (0, pad), (0, 0)))
    go = gt_o_verts
    po = pred_o_verts
    gh_t = jnp.transpose(gh, (0, 2, 1))
    ph_t = jnp.transpose(ph, (0, 2, 1))
    go_t = jnp.transpose(go, (0, 2, 1))
    po_t = jnp.transpose(po, (0, 2, 1))
    a2 = jnp.sum(gh * gh, axis=-1)[:, None, :]
    b2 = jnp.sum(go * go, axis=-1)[:, None, :]
    ght = jnp.concatenate(
        [gh_t, a2, jnp.ones_like(a2), jnp.zeros_like(gh_t)], axis=1)
    bt = jnp.concatenate(
        [-2.0 * go_t, jnp.ones_like(b2), b2, jnp.zeros_like(go_t)], axis=1)
    allh = jnp.stack([gh_t, ph_t], 0)[:, :, :, None, :]
    allo = jnp.stack([go_t, po_t], 0)[:, :, :, None, :]

    rmin2, idxo, cmin2, idxh = _dist_call(ght, bt)
    diffh, diffo = _run_gather(allh, allo, idxo, idxh)
    out = _loss_call(rmin2, cmin2, diffh, diffo)
    return out[0, 0]


# speedup vs baseline: 2.4643x; 1.1253x over previous
import functools

import jax
import jax.numpy as jnp
from jax import lax
from jax.experimental import pallas as pl
from jax.experimental.pallas import tpu as pltpu
from jax.experimental.pallas import tpu_sc as plsc

BT = 8
VH_RAW = 6890
VO = 2048
TH = 128
NT = 54
VH = TH * NT
THR = 0.2
EPS = 1e-8
BIGC = 1e9

_NSLICE = 4
_HS = VH // 3
_OS = VO


def _dist_body(a_ref, bt_ref, rmin_ref, rarg_ref, cmin_ref, carg_ref):
    i = pl.program_id(0)
    exp23 = jnp.int32(0x4B000000)
    sent = jnp.float32(16777216.0)
    m23 = jnp.int32(0x7FFFFF)
    col_iota = lax.bitcast_convert_type(
        lax.broadcasted_iota(jnp.int32, (1, VO), 1) | exp23, jnp.float32)
    row_iota = lax.bitcast_convert_type(
        (lax.broadcasted_iota(jnp.int32, (TH, 1), 0) + i * TH) | exp23,
        jnp.float32)
    rmins, rargs, cmins, cargs = [], [], [], []
    for bb in range(BT):
        a = a_ref[bb]
        bt = bt_ref[bb]
        d2 = lax.dot_general(a, bt, (((0,), (0,)), ((), ())),
                             preferred_element_type=jnp.float32)
        rmin = jnp.min(d2, axis=1)
        rarg_f = jnp.min(
            jnp.where(d2 == rmin[:, None],
                      jnp.broadcast_to(col_iota, (TH, VO)), sent), axis=1)
        rmins.append(rmin)
        rargs.append(lax.bitcast_convert_type(rarg_f, jnp.int32) & m23)
        cmin_t = jnp.min(d2, axis=0)
        carg_f = jnp.min(
            jnp.where(d2 == cmin_t[None, :],
                      jnp.broadcast_to(row_iota, (TH, VO)), sent), axis=0)
        cmins.append(cmin_t)
        cargs.append(lax.bitcast_convert_type(carg_f, jnp.int32) & m23)

    rmin_ref[0] = jnp.stack(rmins)
    rarg_ref[:, 0, :] = jnp.stack(rargs)
    cmin_all = jnp.stack(cmins)
    carg_all = jnp.stack(cargs)

    @pl.when(i == 0)
    def _init():
        cmin_ref[0] = cmin_all
        carg_ref[:, 0, :] = carg_all

    @pl.when(i != 0)
    def _acc():
        prev = cmin_ref[0]
        parg = carg_ref[:, 0, :]
        better = cmin_all < prev
        cmin_ref[0] = jnp.where(better, cmin_all, prev)
        carg_ref[:, 0, :] = jnp.where(better, carg_all, parg)


def _loss_body(d2h_ref, d2o_ref, diffh_ref, diffo_ref, out_ref):
    basis_h = jnp.sqrt(jnp.maximum(d2h_ref[0], 1e-12))
    basis_o = jnp.sqrt(jnp.maximum(d2o_ref[0], 1e-12))
    mask_h = basis_h < THR
    mask_o = basis_o < THR
    sel_h = jnp.sum(mask_h.astype(jnp.int32), axis=-1)
    sel_o = jnp.sum(mask_o.astype(jnp.int32), axis=-1)

    def kth(sel):
        return jnp.maximum(
            1,
            jnp.round(jnp.float32(0.2) * sel.astype(jnp.float32)).astype(
                jnp.int32))

    k_h = kth(sel_h)
    k_o = kth(sel_o)
    bits_h = lax.bitcast_convert_type(basis_h, jnp.int32)
    bits_o = lax.bitcast_convert_type(basis_o, jnp.int32)

    def step(_, carry):
        lo_h, hi_h, lo_o, hi_o = carry

        def halve(lo, hi, mask, bits, k):
            mid = lo + lax.div(hi - lo, 2)
            cnt = jnp.sum(jnp.where(mask & (bits <= mid[..., None]), 1, 0),
                          axis=-1)
            ge = cnt >= k
            return jnp.where(ge, lo, mid + 1), jnp.where(ge, mid, hi)

        lo_h, hi_h = halve(lo_h, hi_h, mask_h, bits_h, k_h)
        lo_o, hi_o = halve(lo_o, hi_o, mask_o, bits_o, k_o)
        return lo_h, hi_h, lo_o, hi_o

    z = jnp.zeros((8,), jnp.int32)
    inf = jnp.full((8,), 0x7F800000, jnp.int32)
    lo_h, _, lo_o, _ = lax.fori_loop(0, 31, step, (z, inf, z, inf))

    def branch(lo, sel, mask, basis, diff_ref):
        t = lax.bitcast_convert_type(lo, jnp.float32)
        t = jnp.where(sel > 0, t, jnp.float32(1.0))[..., None]
        w = jnp.maximum((t - basis) / (t + EPS), 0.0)
        w2 = w * w
        w4 = jnp.where(mask, w2 * w2, 0.0)
        return (jnp.sum(w4[:, None, :] * diff_ref[...])
                / (jnp.sum(w4) + EPS))

    l_h = branch(lo_h, sel_h, mask_h, basis_h, diffh_ref)
    l_o = branch(lo_o, sel_o, mask_o, basis_o, diffo_ref)
    out_ref[...] = jnp.broadcast_to(l_h + l_o, (1, 1))


@functools.lru_cache(maxsize=1)
def _build_gather():
    mesh = plsc.VectorSubcoreMesh(core_axis_name="c", subcore_axis_name="s")
    return functools.partial(
        pl.kernel,
        mesh=mesh,
        out_type=[jax.ShapeDtypeStruct((BT, 1, VH), jnp.float32),
                  jax.ShapeDtypeStruct((BT, 1, VO), jnp.float32)],
        scratch_types=_GATHER_SCRATCH,
        compiler_params=pltpu.CompilerParams(needs_layout_passes=False),
    )(_gather_body)


_GATHER_SCRATCH = [
        pltpu.VMEM((VH,), jnp.float32), pltpu.VMEM((VH,), jnp.float32),
        pltpu.VMEM((VH,), jnp.float32), pltpu.VMEM((VH,), jnp.float32),
        pltpu.VMEM((VH,), jnp.float32), pltpu.VMEM((VH,), jnp.float32),
        pltpu.VMEM((VO,), jnp.float32), pltpu.VMEM((VO,), jnp.float32),
        pltpu.VMEM((VO,), jnp.float32), pltpu.VMEM((VO,), jnp.float32),
        pltpu.VMEM((VO,), jnp.float32), pltpu.VMEM((VO,), jnp.float32),
        pltpu.VMEM((_HS,), jnp.int32), pltpu.VMEM((_OS,), jnp.int32),
        pltpu.VMEM((_HS,), jnp.float32), pltpu.VMEM((_OS,), jnp.float32),
        pltpu.SemaphoreType.DMA,
]


def _gather_body(allh, allo, idxo, idxh, diffh_out, diffo_out,
                 t_ghx, t_ghy, t_ghz, t_phx, t_phy, t_phz,
                 t_gox, t_goy, t_goz, t_pox, t_poy, t_poz,
                 t_idxo, t_idxh, t_dh, t_do, sem):
    wid = lax.axis_index("s") * 2 + lax.axis_index("c")
    b = wid // _NSLICE
    s = wid % _NSLICE
    copies = [
        pltpu.async_copy(allh.at[0, b, 0, 0, :], t_ghx, sem),
        pltpu.async_copy(allh.at[0, b, 1, 0, :], t_ghy, sem),
        pltpu.async_copy(allh.at[0, b, 2, 0, :], t_ghz, sem),
        pltpu.async_copy(allh.at[1, b, 0, 0, :], t_phx, sem),
        pltpu.async_copy(allh.at[1, b, 1, 0, :], t_phy, sem),
        pltpu.async_copy(allh.at[1, b, 2, 0, :], t_phz, sem),
        pltpu.async_copy(allo.at[0, b, 0, 0, :], t_gox, sem),
        pltpu.async_copy(allo.at[0, b, 1, 0, :], t_goy, sem),
        pltpu.async_copy(allo.at[0, b, 2, 0, :], t_goz, sem),
        pltpu.async_copy(allo.at[1, b, 0, 0, :], t_pox, sem),
        pltpu.async_copy(allo.at[1, b, 1, 0, :], t_poy, sem),
        pltpu.async_copy(allo.at[1, b, 2, 0, :], t_poz, sem),
    ]

    @pl.when(s < 3)
    def _hside():
        hoff = s * _HS
        pltpu.sync_copy(idxo.at[b, 0, pl.ds(hoff, _HS)], t_idxo)
        for c in copies:
            c.wait()

        def hstep(c, carry):
            for u in range(2):
                base = c * 32 + u * 16
                idx = t_idxo[pl.ds(base, 16)]
                gx = plsc.load_gather(t_gox, [idx])
                gy = plsc.load_gather(t_goy, [idx])
                gz = plsc.load_gather(t_goz, [idx])
                px = plsc.load_gather(t_pox, [idx])
                py = plsc.load_gather(t_poy, [idx])
                pz = plsc.load_gather(t_poz, [idx])
                o = hoff + base
                dx = (px - t_phx[pl.ds(o, 16)]) - (gx - t_ghx[pl.ds(o, 16)])
                dy = (py - t_phy[pl.ds(o, 16)]) - (gy - t_ghy[pl.ds(o, 16)])
                dz = (pz - t_phz[pl.ds(o, 16)]) - (gz - t_ghz[pl.ds(o, 16)])
                t_dh[pl.ds(base, 16)] = (jnp.abs(dx) + jnp.abs(dy)
                                         + jnp.abs(dz))
            return carry

        lax.fori_loop(0, _HS // 32, hstep, 0)
        pltpu.sync_copy(t_dh, diffh_out.at[b, 0, pl.ds(hoff, _HS)])

    @pl.when(s == 3)
    def _oside():
        pltpu.sync_copy(idxh.at[b, 0, :], t_idxh)
        for c in copies:
            c.wait()

        def ostep(c, carry):
            for u in range(2):
                base = c * 32 + u * 16
                idx = t_idxh[pl.ds(base, 16)]
                gx = plsc.load_gather(t_ghx, [idx])
                gy = plsc.load_gather(t_ghy, [idx])
                gz = plsc.load_gather(t_ghz, [idx])
                px = plsc.load_gather(t_phx, [idx])
                py = plsc.load_gather(t_phy, [idx])
                pz = plsc.load_gather(t_phz, [idx])
                dx = ((px - t_pox[pl.ds(base, 16)])
                      - (gx - t_gox[pl.ds(base, 16)]))
                dy = ((py - t_poy[pl.ds(base, 16)])
                      - (gy - t_goy[pl.ds(base, 16)]))
                dz = ((pz - t_poz[pl.ds(base, 16)])
                      - (gz - t_goz[pl.ds(base, 16)]))
                t_do[pl.ds(base, 16)] = (jnp.abs(dx) + jnp.abs(dy)
                                         + jnp.abs(dz))
            return carry

        lax.fori_loop(0, _OS // 32, ostep, 0)
        pltpu.sync_copy(t_do, diffo_out.at[b, 0, :])


def _dist_call(ght, bt):
    return pl.pallas_call(
        _dist_body,
        grid=(NT,),
        in_specs=[
            pl.BlockSpec((BT, 8, TH), lambda i: (0, 0, i)),
            pl.BlockSpec((BT, 8, VO), lambda i: (0, 0, 0)),
        ],
        out_specs=[
            pl.BlockSpec((1, BT, TH), lambda i: (0, 0, i)),
            pl.BlockSpec((BT, 1, TH), lambda i: (0, 0, i)),
            pl.BlockSpec((1, BT, VO), lambda i: (0, 0, 0)),
            pl.BlockSpec((BT, 1, VO), lambda i: (0, 0, 0)),
        ],
        out_shape=[
            jax.ShapeDtypeStruct((1, BT, VH), jnp.float32),
            jax.ShapeDtypeStruct((BT, 1, VH), jnp.int32),
            jax.ShapeDtypeStruct((1, BT, VO), jnp.float32),
            jax.ShapeDtypeStruct((BT, 1, VO), jnp.int32),
        ],
        compiler_params=pltpu.CompilerParams(
            dimension_semantics=("arbitrary",),
            fuse_transposed_lhs_in_matmul=True),
    )(ght, bt)


def _loss_call(d2h, d2o, diffh, diffo):
    return pl.pallas_call(
        _loss_body,
        out_shape=jax.ShapeDtypeStruct((1, 1), jnp.float32),
    )(d2h, d2o, diffh, diffo)


def _run_gather(allh, allo, idxo, idxh):
    return _build_gather()(allh, allo, idxo, idxh)


def kernel(pred_h_verts, pred_o_verts, gt_h_verts, gt_o_verts):
    pad = VH - VH_RAW
    gh = jnp.pad(gt_h_verts, ((0, 0), (0, pad), (0, 0)), constant_values=BIGC)
    ph = jnp.pad(pred_h_verts, ((0, 0), (0, pad), (0, 0)))
    go = gt_o_verts
    po = pred_o_verts
    gh_t = jnp.transpose(gh, (0, 2, 1))
    ph_t = jnp.transpose(ph, (0, 2, 1))
    go_t = jnp.transpose(go, (0, 2, 1))
    po_t = jnp.transpose(po, (0, 2, 1))
    a2 = jnp.sum(gh * gh, axis=-1)[:, None, :]
    b2 = jnp.sum(go * go, axis=-1)[:, None, :]
    ght = jnp.concatenate(
        [gh_t, a2, jnp.ones_like(a2), jnp.zeros_like(gh_t)], axis=1)
    bt = jnp.concatenate(
        [-2.0 * go_t, jnp.ones_like(b2), b2, jnp.zeros_like(go_t)], axis=1)
    allh = jnp.stack([gh_t, ph_t], 0)[:, :, :, None, :]
    allo = jnp.stack([go_t, po_t], 0)[:, :, :, None, :]

    rmin2, idxo, cmin2, idxh = _dist_call(ght, bt)
    diffh, diffo = _run_gather(allh, allo, idxo, idxh)
    out = _loss_call(rmin2, cmin2, diffh, diffo)
    return out[0, 0]


# speedup vs baseline: 2.6614x; 1.0800x over previous
import functools

import jax
import jax.numpy as jnp
from jax import lax
from jax.experimental import pallas as pl
from jax.experimental.pallas import tpu as pltpu
from jax.experimental.pallas import tpu_sc as plsc

BT = 8
VH_RAW = 6890
VO = 2048
TH = 256
NT = 27
VH = TH * NT
THR = 0.2
EPS = 1e-8
BIGC = 1e9

_NSLICE = 4
_HS = VH // 3
_OS = VO


def _dist_body(a_ref, bt_ref, rmin_ref, rarg_ref, cmin_ref, carg_ref):
    i = pl.program_id(0)
    exp23 = jnp.int32(0x4B000000)
    sent = jnp.float32(16777216.0)
    m23 = jnp.int32(0x7FFFFF)
    col_iota = lax.bitcast_convert_type(
        lax.broadcasted_iota(jnp.int32, (1, VO), 1) | exp23, jnp.float32)
    row_iota = lax.bitcast_convert_type(
        (lax.broadcasted_iota(jnp.int32, (TH, 1), 0) + i * TH) | exp23,
        jnp.float32)
    rmins, rargs, cmins, cargs = [], [], [], []
    for bb in range(BT):
        a = a_ref[bb]
        bt = bt_ref[bb]
        d2 = lax.dot_general(a, bt, (((0,), (0,)), ((), ())),
                             preferred_element_type=jnp.float32)
        rmin = jnp.min(d2, axis=1)
        rarg_f = jnp.min(
            jnp.where(d2 == rmin[:, None],
                      jnp.broadcast_to(col_iota, (TH, VO)), sent), axis=1)
        rmins.append(rmin)
        rargs.append(lax.bitcast_convert_type(rarg_f, jnp.int32) & m23)
        cmin_t = jnp.min(d2, axis=0)
        carg_f = jnp.min(
            jnp.where(d2 == cmin_t[None, :],
                      jnp.broadcast_to(row_iota, (TH, VO)), sent), axis=0)
        cmins.append(cmin_t)
        cargs.append(lax.bitcast_convert_type(carg_f, jnp.int32) & m23)

    rmin_ref[0] = jnp.stack(rmins)
    rarg_ref[:, 0, :] = jnp.stack(rargs)
    cmin_all = jnp.stack(cmins)
    carg_all = jnp.stack(cargs)

    @pl.when(i == 0)
    def _init():
        cmin_ref[0] = cmin_all
        carg_ref[:, 0, :] = carg_all

    @pl.when(i != 0)
    def _acc():
        prev = cmin_ref[0]
        parg = carg_ref[:, 0, :]
        better = cmin_all < prev
        cmin_ref[0] = jnp.where(better, cmin_all, prev)
        carg_ref[:, 0, :] = jnp.where(better, carg_all, parg)


def _loss_body(d2h_ref, d2o_ref, diffh_ref, diffo_ref, out_ref):
    basis_h = jnp.sqrt(jnp.maximum(d2h_ref[0], 1e-12))
    basis_o = jnp.sqrt(jnp.maximum(d2o_ref[0], 1e-12))
    mask_h = basis_h < THR
    mask_o = basis_o < THR
    sel_h = jnp.sum(mask_h.astype(jnp.int32), axis=-1)
    sel_o = jnp.sum(mask_o.astype(jnp.int32), axis=-1)

    def kth(sel):
        return jnp.maximum(
            1,
            jnp.round(jnp.float32(0.2) * sel.astype(jnp.float32)).astype(
                jnp.int32))

    k_h = kth(sel_h)
    k_o = kth(sel_o)
    bits_h = lax.bitcast_convert_type(basis_h, jnp.int32)
    bits_o = lax.bitcast_convert_type(basis_o, jnp.int32)

    def step(_, carry):
        lo_h, hi_h, lo_o, hi_o = carry

        def halve(lo, hi, mask, bits, k):
            mid = lo + lax.div(hi - lo, 2)
            cnt = jnp.sum(jnp.where(mask & (bits <= mid[..., None]), 1, 0),
                          axis=-1)
            ge = cnt >= k
            return jnp.where(ge, lo, mid + 1), jnp.where(ge, mid, hi)

        lo_h, hi_h = halve(lo_h, hi_h, mask_h, bits_h, k_h)
        lo_o, hi_o = halve(lo_o, hi_o, mask_o, bits_o, k_o)
        return lo_h, hi_h, lo_o, hi_o

    z = jnp.zeros((8,), jnp.int32)
    inf = jnp.full((8,), 0x7F800000, jnp.int32)
    lo_h, _, lo_o, _ = lax.fori_loop(0, 31, step, (z, inf, z, inf))

    def branch(lo, sel, mask, basis, diff_ref):
        t = lax.bitcast_convert_type(lo, jnp.float32)
        t = jnp.where(sel > 0, t, jnp.float32(1.0))[..., None]
        w = jnp.maximum((t - basis) / (t + EPS), 0.0)
        w2 = w * w
        w4 = jnp.where(mask, w2 * w2, 0.0)
        return (jnp.sum(w4[:, None, :] * diff_ref[...])
                / (jnp.sum(w4) + EPS))

    l_h = branch(lo_h, sel_h, mask_h, basis_h, diffh_ref)
    l_o = branch(lo_o, sel_o, mask_o, basis_o, diffo_ref)
    out_ref[...] = jnp.broadcast_to(l_h + l_o, (1, 1))


@functools.lru_cache(maxsize=1)
def _build_gather():
    mesh = plsc.VectorSubcoreMesh(core_axis_name="c", subcore_axis_name="s")
    return functools.partial(
        pl.kernel,
        mesh=mesh,
        out_type=[jax.ShapeDtypeStruct((BT, 1, VH), jnp.float32),
                  jax.ShapeDtypeStruct((BT, 1, VO), jnp.float32)],
        scratch_types=_GATHER_SCRATCH,
        compiler_params=pltpu.CompilerParams(needs_layout_passes=False),
    )(_gather_body)


_GATHER_SCRATCH = [
        pltpu.VMEM((VH,), jnp.float32), pltpu.VMEM((VH,), jnp.float32),
        pltpu.VMEM((VH,), jnp.float32), pltpu.VMEM((VH,), jnp.float32),
        pltpu.VMEM((VH,), jnp.float32), pltpu.VMEM((VH,), jnp.float32),
        pltpu.VMEM((VO,), jnp.float32), pltpu.VMEM((VO,), jnp.float32),
        pltpu.VMEM((VO,), jnp.float32), pltpu.VMEM((VO,), jnp.float32),
        pltpu.VMEM((VO,), jnp.float32), pltpu.VMEM((VO,), jnp.float32),
        pltpu.VMEM((_HS,), jnp.int32), pltpu.VMEM((_OS,), jnp.int32),
        pltpu.VMEM((_HS,), jnp.float32), pltpu.VMEM((_OS,), jnp.float32),
        pltpu.SemaphoreType.DMA,
]


def _gather_body(allh, allo, idxo, idxh, diffh_out, diffo_out,
                 t_ghx, t_ghy, t_ghz, t_phx, t_phy, t_phz,
                 t_gox, t_goy, t_goz, t_pox, t_poy, t_poz,
                 t_idxo, t_idxh, t_dh, t_do, sem):
    wid = lax.axis_index("s") * 2 + lax.axis_index("c")
    b = wid // _NSLICE
    s = wid % _NSLICE
    copies = [
        pltpu.async_copy(allh.at[0, b, 0, 0, :], t_ghx, sem),
        pltpu.async_copy(allh.at[0, b, 1, 0, :], t_ghy, sem),
        pltpu.async_copy(allh.at[0, b, 2, 0, :], t_ghz, sem),
        pltpu.async_copy(allh.at[1, b, 0, 0, :], t_phx, sem),
        pltpu.async_copy(allh.at[1, b, 1, 0, :], t_phy, sem),
        pltpu.async_copy(allh.at[1, b, 2, 0, :], t_phz, sem),
        pltpu.async_copy(allo.at[0, b, 0, 0, :], t_gox, sem),
        pltpu.async_copy(allo.at[0, b, 1, 0, :], t_goy, sem),
        pltpu.async_copy(allo.at[0, b, 2, 0, :], t_goz, sem),
        pltpu.async_copy(allo.at[1, b, 0, 0, :], t_pox, sem),
        pltpu.async_copy(allo.at[1, b, 1, 0, :], t_poy, sem),
        pltpu.async_copy(allo.at[1, b, 2, 0, :], t_poz, sem),
    ]

    @pl.when(s < 3)
    def _hside():
        hoff = s * _HS
        pltpu.sync_copy(idxo.at[b, 0, pl.ds(hoff, _HS)], t_idxo)
        for c in copies:
            c.wait()

        def hstep(c, carry):
            for u in range(2):
                base = c * 32 + u * 16
                idx = t_idxo[pl.ds(base, 16)]
                gx = plsc.load_gather(t_gox, [idx])
                gy = plsc.load_gather(t_goy, [idx])
                gz = plsc.load_gather(t_goz, [idx])
                px = plsc.load_gather(t_pox, [idx])
                py = plsc.load_gather(t_poy, [idx])
                pz = plsc.load_gather(t_poz, [idx])
                o = hoff + base
                dx = (px - t_phx[pl.ds(o, 16)]) - (gx - t_ghx[pl.ds(o, 16)])
                dy = (py - t_phy[pl.ds(o, 16)]) - (gy - t_ghy[pl.ds(o, 16)])
                dz = (pz - t_phz[pl.ds(o, 16)]) - (gz - t_ghz[pl.ds(o, 16)])
                t_dh[pl.ds(base, 16)] = (jnp.abs(dx) + jnp.abs(dy)
                                         + jnp.abs(dz))
            return carry

        lax.fori_loop(0, _HS // 32, hstep, 0)
        pltpu.sync_copy(t_dh, diffh_out.at[b, 0, pl.ds(hoff, _HS)])

    @pl.when(s == 3)
    def _oside():
        pltpu.sync_copy(idxh.at[b, 0, :], t_idxh)
        for c in copies:
            c.wait()

        def ostep(c, carry):
            for u in range(2):
                base = c * 32 + u * 16
                idx = t_idxh[pl.ds(base, 16)]
                gx = plsc.load_gather(t_ghx, [idx])
                gy = plsc.load_gather(t_ghy, [idx])
                gz = plsc.load_gather(t_ghz, [idx])
                px = plsc.load_gather(t_phx, [idx])
                py = plsc.load_gather(t_phy, [idx])
                pz = plsc.load_gather(t_phz, [idx])
                dx = ((px - t_pox[pl.ds(base, 16)])
                      - (gx - t_gox[pl.ds(base, 16)]))
                dy = ((py - t_poy[pl.ds(base, 16)])
                      - (gy - t_goy[pl.ds(base, 16)]))
                dz = ((pz - t_poz[pl.ds(base, 16)])
                      - (gz - t_goz[pl.ds(base, 16)]))
                t_do[pl.ds(base, 16)] = (jnp.abs(dx) + jnp.abs(dy)
                                         + jnp.abs(dz))
            return carry

        lax.fori_loop(0, _OS // 32, ostep, 0)
        pltpu.sync_copy(t_do, diffo_out.at[b, 0, :])


def _dist_call(ght, bt):
    return pl.pallas_call(
        _dist_body,
        grid=(NT,),
        in_specs=[
            pl.BlockSpec((BT, 8, TH), lambda i: (0, 0, i)),
            pl.BlockSpec((BT, 8, VO), lambda i: (0, 0, 0)),
        ],
        out_specs=[
            pl.BlockSpec((1, BT, TH), lambda i: (0, 0, i)),
            pl.BlockSpec((BT, 1, TH), lambda i: (0, 0, i)),
            pl.BlockSpec((1, BT, VO), lambda i: (0, 0, 0)),
            pl.BlockSpec((BT, 1, VO), lambda i: (0, 0, 0)),
        ],
        out_shape=[
            jax.ShapeDtypeStruct((1, BT, VH), jnp.float32),
            jax.ShapeDtypeStruct((BT, 1, VH), jnp.int32),
            jax.ShapeDtypeStruct((1, BT, VO), jnp.float32),
            jax.ShapeDtypeStruct((BT, 1, VO), jnp.int32),
        ],
        compiler_params=pltpu.CompilerParams(
            dimension_semantics=("arbitrary",),
            fuse_transposed_lhs_in_matmul=True),
    )(ght, bt)


def _loss_call(d2h, d2o, diffh, diffo):
    return pl.pallas_call(
        _loss_body,
        out_shape=jax.ShapeDtypeStruct((1, 1), jnp.float32),
    )(d2h, d2o, diffh, diffo)


def _run_gather(allh, allo, idxo, idxh):
    return _build_gather()(allh, allo, idxo, idxh)


def kernel(pred_h_verts, pred_o_verts, gt_h_verts, gt_o_verts):
    pad = VH - VH_RAW
    gh = jnp.pad(gt_h_verts, ((0, 0), (0, pad), (0, 0)), constant_values=BIGC)
    ph = jnp.pad(pred_h_verts, ((0, 0), (0, pad), (0, 0)))
    go = gt_o_verts
    po = pred_o_verts
    gh_t = jnp.transpose(gh, (0, 2, 1))
    ph_t = jnp.transpose(ph, (0, 2, 1))
    go_t = jnp.transpose(go, (0, 2, 1))
    po_t = jnp.transpose(po, (0, 2, 1))
    a2 = jnp.sum(gh * gh, axis=-1)[:, None, :]
    b2 = jnp.sum(go * go, axis=-1)[:, None, :]
    ght = jnp.concatenate(
        [gh_t, a2, jnp.ones_like(a2), jnp.zeros_like(gh_t)], axis=1)
    bt = jnp.concatenate(
        [-2.0 * go_t, jnp.ones_like(b2), b2, jnp.zeros_like(go_t)], axis=1)
    allh = jnp.stack([gh_t, ph_t], 0)[:, :, :, None, :]
    allo = jnp.stack([go_t, po_t], 0)[:, :, :, None, :]

    rmin2, idxo, cmin2, idxh = _dist_call(ght, bt)
    diffh, diffo = _run_gather(allh, allo, idxo, idxh)
    out = _loss_call(rmin2, cmin2, diffh, diffo)
    return out[0, 0]


# speedup vs baseline: 2.6670x; 1.0021x over previous
import functools

import jax
import jax.numpy as jnp
from jax import lax
from jax.experimental import pallas as pl
from jax.experimental.pallas import tpu as pltpu
from jax.experimental.pallas import tpu_sc as plsc

BT = 8
VH_RAW = 6890
VO = 2048
TH = 384
NT = 18
VH = TH * NT
THR = 0.2
EPS = 1e-8
BIGC = 1e9

_NSLICE = 4
_HS = VH // 3
_OS = VO


def _dist_body(a_ref, bt_ref, rmin_ref, rarg_ref, cmin_ref, carg_ref):
    i = pl.program_id(0)
    exp23 = jnp.int32(0x4B000000)
    sent = jnp.float32(16777216.0)
    m23 = jnp.int32(0x7FFFFF)
    col_iota = lax.bitcast_convert_type(
        lax.broadcasted_iota(jnp.int32, (1, VO), 1) | exp23, jnp.float32)
    row_iota = lax.bitcast_convert_type(
        (lax.broadcasted_iota(jnp.int32, (TH, 1), 0) + i * TH) | exp23,
        jnp.float32)
    rmins, rargs, cmins, cargs = [], [], [], []
    for bb in range(BT):
        a = a_ref[bb]
        bt = bt_ref[bb]
        d2 = lax.dot_general(a, bt, (((0,), (0,)), ((), ())),
                             preferred_element_type=jnp.float32)
        rmin = jnp.min(d2, axis=1)
        rarg_f = jnp.min(
            jnp.where(d2 == rmin[:, None],
                      jnp.broadcast_to(col_iota, (TH, VO)), sent), axis=1)
        rmins.append(rmin)
        rargs.append(lax.bitcast_convert_type(rarg_f, jnp.int32) & m23)
        cmin_t = jnp.min(d2, axis=0)
        carg_f = jnp.min(
            jnp.where(d2 == cmin_t[None, :],
                      jnp.broadcast_to(row_iota, (TH, VO)), sent), axis=0)
        cmins.append(cmin_t)
        cargs.append(lax.bitcast_convert_type(carg_f, jnp.int32) & m23)

    rmin_ref[0] = jnp.stack(rmins)
    rarg_ref[:, 0, :] = jnp.stack(rargs)
    cmin_all = jnp.stack(cmins)
    carg_all = jnp.stack(cargs)

    @pl.when(i == 0)
    def _init():
        cmin_ref[0] = cmin_all
        carg_ref[:, 0, :] = carg_all

    @pl.when(i != 0)
    def _acc():
        prev = cmin_ref[0]
        parg = carg_ref[:, 0, :]
        better = cmin_all < prev
        cmin_ref[0] = jnp.where(better, cmin_all, prev)
        carg_ref[:, 0, :] = jnp.where(better, carg_all, parg)


def _loss_body(d2h_ref, d2o_ref, diffh_ref, diffo_ref, out_ref):
    basis_h = jnp.sqrt(jnp.maximum(d2h_ref[0], 1e-12))
    basis_o = jnp.sqrt(jnp.maximum(d2o_ref[0], 1e-12))
    mask_h = basis_h < THR
    mask_o = basis_o < THR
    sel_h = jnp.sum(mask_h.astype(jnp.int32), axis=-1)
    sel_o = jnp.sum(mask_o.astype(jnp.int32), axis=-1)

    def kth(sel):
        return jnp.maximum(
            1,
            jnp.round(jnp.float32(0.2) * sel.astype(jnp.float32)).astype(
                jnp.int32))

    k_h = kth(sel_h)
    k_o = kth(sel_o)
    bits_h = lax.bitcast_convert_type(basis_h, jnp.int32)
    bits_o = lax.bitcast_convert_type(basis_o, jnp.int32)

    def step(_, carry):
        lo_h, hi_h, lo_o, hi_o = carry

        def halve(lo, hi, mask, bits, k):
            mid = lo + lax.div(hi - lo, 2)
            cnt = jnp.sum(jnp.where(mask & (bits <= mid[..., None]), 1, 0),
                          axis=-1)
            ge = cnt >= k
            return jnp.where(ge, lo, mid + 1), jnp.where(ge, mid, hi)

        lo_h, hi_h = halve(lo_h, hi_h, mask_h, bits_h, k_h)
        lo_o, hi_o = halve(lo_o, hi_o, mask_o, bits_o, k_o)
        return lo_h, hi_h, lo_o, hi_o

    z = jnp.zeros((8,), jnp.int32)
    inf = jnp.full((8,), 0x7F800000, jnp.int32)
    lo_h, _, lo_o, _ = lax.fori_loop(0, 31, step, (z, inf, z, inf))

    def branch(lo, sel, mask, basis, diff_ref):
        t = lax.bitcast_convert_type(lo, jnp.float32)
        t = jnp.where(sel > 0, t, jnp.float32(1.0))[..., None]
        w = jnp.maximum((t - basis) / (t + EPS), 0.0)
        w2 = w * w
        w4 = jnp.where(mask, w2 * w2, 0.0)
        return (jnp.sum(w4[:, None, :] * diff_ref[...])
                / (jnp.sum(w4) + EPS))

    l_h = branch(lo_h, sel_h, mask_h, basis_h, diffh_ref)
    l_o = branch(lo_o, sel_o, mask_o, basis_o, diffo_ref)
    out_ref[...] = jnp.broadcast_to(l_h + l_o, (1, 1))


@functools.lru_cache(maxsize=1)
def _build_gather():
    mesh = plsc.VectorSubcoreMesh(core_axis_name="c", subcore_axis_name="s")
    return functools.partial(
        pl.kernel,
        mesh=mesh,
        out_type=[jax.ShapeDtypeStruct((BT, 1, VH), jnp.float32),
                  jax.ShapeDtypeStruct((BT, 1, VO), jnp.float32)],
        scratch_types=_GATHER_SCRATCH,
        compiler_params=pltpu.CompilerParams(needs_layout_passes=False),
    )(_gather_body)


_GATHER_SCRATCH = [
        pltpu.VMEM((VH,), jnp.float32), pltpu.VMEM((VH,), jnp.float32),
        pltpu.VMEM((VH,), jnp.float32), pltpu.VMEM((VH,), jnp.float32),
        pltpu.VMEM((VH,), jnp.float32), pltpu.VMEM((VH,), jnp.float32),
        pltpu.VMEM((VO,), jnp.float32), pltpu.VMEM((VO,), jnp.float32),
        pltpu.VMEM((VO,), jnp.float32), pltpu.VMEM((VO,), jnp.float32),
        pltpu.VMEM((VO,), jnp.float32), pltpu.VMEM((VO,), jnp.float32),
        pltpu.VMEM((_HS,), jnp.int32), pltpu.VMEM((_OS,), jnp.int32),
        pltpu.VMEM((_HS,), jnp.float32), pltpu.VMEM((_OS,), jnp.float32),
        pltpu.SemaphoreType.DMA,
]


def _gather_body(allh, allo, idxo, idxh, diffh_out, diffo_out,
                 t_ghx, t_ghy, t_ghz, t_phx, t_phy, t_phz,
                 t_gox, t_goy, t_goz, t_pox, t_poy, t_poz,
                 t_idxo, t_idxh, t_dh, t_do, sem):
    wid = lax.axis_index("s") * 2 + lax.axis_index("c")
    b = wid // _NSLICE
    s = wid % _NSLICE
    copies = [
        pltpu.async_copy(allh.at[0, b, 0, 0, :], t_ghx, sem),
        pltpu.async_copy(allh.at[0, b, 1, 0, :], t_ghy, sem),
        pltpu.async_copy(allh.at[0, b, 2, 0, :], t_ghz, sem),
        pltpu.async_copy(allh.at[1, b, 0, 0, :], t_phx, sem),
        pltpu.async_copy(allh.at[1, b, 1, 0, :], t_phy, sem),
        pltpu.async_copy(allh.at[1, b, 2, 0, :], t_phz, sem),
        pltpu.async_copy(allo.at[0, b, 0, 0, :], t_gox, sem),
        pltpu.async_copy(allo.at[0, b, 1, 0, :], t_goy, sem),
        pltpu.async_copy(allo.at[0, b, 2, 0, :], t_goz, sem),
        pltpu.async_copy(allo.at[1, b, 0, 0, :], t_pox, sem),
        pltpu.async_copy(allo.at[1, b, 1, 0, :], t_poy, sem),
        pltpu.async_copy(allo.at[1, b, 2, 0, :], t_poz, sem),
    ]

    @pl.when(s < 3)
    def _hside():
        hoff = s * _HS
        pltpu.sync_copy(idxo.at[b, 0, pl.ds(hoff, _HS)], t_idxo)
        for c in copies:
            c.wait()

        def hstep(c, carry):
            for u in range(2):
                base = c * 32 + u * 16
                idx = t_idxo[pl.ds(base, 16)]
                gx = plsc.load_gather(t_gox, [idx])
                gy = plsc.load_gather(t_goy, [idx])
                gz = plsc.load_gather(t_goz, [idx])
                px = plsc.load_gather(t_pox, [idx])
                py = plsc.load_gather(t_poy, [idx])
                pz = plsc.load_gather(t_poz, [idx])
                o = hoff + base
                dx = (px - t_phx[pl.ds(o, 16)]) - (gx - t_ghx[pl.ds(o, 16)])
                dy = (py - t_phy[pl.ds(o, 16)]) - (gy - t_ghy[pl.ds(o, 16)])
                dz = (pz - t_phz[pl.ds(o, 16)]) - (gz - t_ghz[pl.ds(o, 16)])
                t_dh[pl.ds(base, 16)] = (jnp.abs(dx) + jnp.abs(dy)
                                         + jnp.abs(dz))
            return carry

        lax.fori_loop(0, _HS // 32, hstep, 0)
        pltpu.sync_copy(t_dh, diffh_out.at[b, 0, pl.ds(hoff, _HS)])

    @pl.when(s == 3)
    def _oside():
        pltpu.sync_copy(idxh.at[b, 0, :], t_idxh)
        for c in copies:
            c.wait()

        def ostep(c, carry):
            for u in range(2):
                base = c * 32 + u * 16
                idx = t_idxh[pl.ds(base, 16)]
                gx = plsc.load_gather(t_ghx, [idx])
                gy = plsc.load_gather(t_ghy, [idx])
                gz = plsc.load_gather(t_ghz, [idx])
                px = plsc.load_gather(t_phx, [idx])
                py = plsc.load_gather(t_phy, [idx])
                pz = plsc.load_gather(t_phz, [idx])
                dx = ((px - t_pox[pl.ds(base, 16)])
                      - (gx - t_gox[pl.ds(base, 16)]))
                dy = ((py - t_poy[pl.ds(base, 16)])
                      - (gy - t_goy[pl.ds(base, 16)]))
                dz = ((pz - t_poz[pl.ds(base, 16)])
                      - (gz - t_goz[pl.ds(base, 16)]))
                t_do[pl.ds(base, 16)] = (jnp.abs(dx) + jnp.abs(dy)
                                         + jnp.abs(dz))
            return carry

        lax.fori_loop(0, _OS // 32, ostep, 0)
        pltpu.sync_copy(t_do, diffo_out.at[b, 0, :])


def _dist_call(ght, bt):
    return pl.pallas_call(
        _dist_body,
        grid=(NT,),
        in_specs=[
            pl.BlockSpec((BT, 8, TH), lambda i: (0, 0, i)),
            pl.BlockSpec((BT, 8, VO), lambda i: (0, 0, 0)),
        ],
        out_specs=[
            pl.BlockSpec((1, BT, TH), lambda i: (0, 0, i)),
            pl.BlockSpec((BT, 1, TH), lambda i: (0, 0, i)),
            pl.BlockSpec((1, BT, VO), lambda i: (0, 0, 0)),
            pl.BlockSpec((BT, 1, VO), lambda i: (0, 0, 0)),
        ],
        out_shape=[
            jax.ShapeDtypeStruct((1, BT, VH), jnp.float32),
            jax.ShapeDtypeStruct((BT, 1, VH), jnp.int32),
            jax.ShapeDtypeStruct((1, BT, VO), jnp.float32),
            jax.ShapeDtypeStruct((BT, 1, VO), jnp.int32),
        ],
        compiler_params=pltpu.CompilerParams(
            dimension_semantics=("arbitrary",),
            fuse_transposed_lhs_in_matmul=True),
    )(ght, bt)


def _loss_call(d2h, d2o, diffh, diffo):
    return pl.pallas_call(
        _loss_body,
        out_shape=jax.ShapeDtypeStruct((1, 1), jnp.float32),
    )(d2h, d2o, diffh, diffo)


def _run_gather(allh, allo, idxo, idxh):
    return _build_gather()(allh, allo, idxo, idxh)


def kernel(pred_h_verts, pred_o_verts, gt_h_verts, gt_o_verts):
    pad = VH - VH_RAW
    gh = jnp.pad(gt_h_verts, ((0, 0), (0, pad), (0, 0)), constant_values=BIGC)
    ph = jnp.pad(pred_h_verts, ((0, 0), (0, pad), (0, 0)))
    go = gt_o_verts
    po = pred_o_verts
    gh_t = jnp.transpose(gh, (0, 2, 1))
    ph_t = jnp.transpose(ph, (0, 2, 1))
    go_t = jnp.transpose(go, (0, 2, 1))
    po_t = jnp.transpose(po, (0, 2, 1))
    a2 = jnp.sum(gh * gh, axis=-1)[:, None, :]
    b2 = jnp.sum(go * go, axis=-1)[:, None, :]
    ght = jnp.concatenate(
        [gh_t, a2, jnp.ones_like(a2), jnp.zeros_like(gh_t)], axis=1)
    bt = jnp.concatenate(
        [-2.0 * go_t, jnp.ones_like(b2), b2, jnp.zeros_like(go_t)], axis=1)
    allh = jnp.stack([gh_t, ph_t], 0)[:, :, :, None, :]
    allo = jnp.stack([go_t, po_t], 0)[:, :, :, None, :]

    rmin2, idxo, cmin2, idxh = _dist_call(ght, bt)
    diffh, diffo = _run_gather(allh, allo, idxo, idxh)
    out = _loss_call(rmin2, cmin2, diffh, diffo)
    return out[0, 0]


# speedup vs baseline: 3.2832x; 1.2311x over previous
import functools

import jax
import jax.numpy as jnp
from jax import lax
from jax.experimental import pallas as pl
from jax.experimental.pallas import tpu as pltpu
from jax.experimental.pallas import tpu_sc as plsc

BT = 8
VH_RAW = 6890
VO = 2048
TH = 384
NT = 18
VH = TH * NT
THR = 0.2
EPS = 1e-8
BIGC = 1e9

_NSLICE = 4
_HS = VH // 3
_OS = VO


def _dist_body(a_ref, bt_ref, rmin_ref, rarg_ref, cmin_ref, carg_ref):
    i = pl.program_id(0)
    nch = VO // 128
    nst = TH // 8
    lane_f = lax.broadcasted_iota(
        jnp.int32, (1, 128), 1).astype(jnp.float32)
    sub_f = lax.broadcasted_iota(jnp.int32, (8, 1), 0).astype(jnp.float32)
    rmins, rargs, cmins, cargs = [], [], [], []
    for bb in range(BT):
        a = a_ref[bb]
        bt = bt_ref[bb]
        d2 = lax.dot_general(a, bt, (((0,), (0,)), ((), ())),
                             preferred_element_type=jnp.float32)
        run_min = d2[:, 0:128]
        run_ci = jnp.zeros((TH, 128), jnp.float32)
        for c in range(1, nch):
            ch = d2[:, c * 128:(c + 1) * 128]
            lt = ch < run_min
            run_min = jnp.minimum(run_min, ch)
            run_ci = jnp.where(lt, jnp.float32(c), run_ci)
        rmin = jnp.min(run_min, axis=1)
        enc = run_ci * 128.0 + jnp.broadcast_to(lane_f, (TH, 128))
        cand = jnp.where(run_min == rmin[:, None], enc, jnp.float32(4096.0))
        rmins.append(rmin)
        rargs.append(jnp.min(cand, axis=1).astype(jnp.int32))
        run_cmin = d2[0:8, :]
        run_si = jnp.zeros((8, VO), jnp.float32)
        for s_ in range(1, nst):
            st = d2[s_ * 8:(s_ + 1) * 8, :]
            lt = st < run_cmin
            run_cmin = jnp.minimum(run_cmin, st)
            run_si = jnp.where(lt, jnp.float32(s_), run_si)
        cmin_t = jnp.min(run_cmin, axis=0)
        rowenc = (run_si * 8.0 + jnp.broadcast_to(sub_f, (8, VO))
                  + (i * TH).astype(jnp.float32))
        ccand = jnp.where(run_cmin == cmin_t[None, :], rowenc,
                          jnp.float32(16384.0))
        cmins.append(cmin_t)
        cargs.append(jnp.min(ccand, axis=0).astype(jnp.int32))

    rmin_ref[0] = jnp.stack(rmins)
    rarg_ref[:, 0, :] = jnp.stack(rargs)
    cmin_all = jnp.stack(cmins)
    carg_all = jnp.stack(cargs)

    @pl.when(i == 0)
    def _init():
        cmin_ref[0] = cmin_all
        carg_ref[:, 0, :] = carg_all

    @pl.when(i != 0)
    def _acc():
        prev = cmin_ref[0]
        parg = carg_ref[:, 0, :]
        better = cmin_all < prev
        cmin_ref[0] = jnp.where(better, cmin_all, prev)
        carg_ref[:, 0, :] = jnp.where(better, carg_all, parg)


def _loss_body(d2h_ref, d2o_ref, diffh_ref, diffo_ref, out_ref):
    basis_h = jnp.sqrt(jnp.maximum(d2h_ref[0], 1e-12))
    basis_o = jnp.sqrt(jnp.maximum(d2o_ref[0], 1e-12))
    mask_h = basis_h < THR
    mask_o = basis_o < THR
    sel_h = jnp.sum(mask_h.astype(jnp.int32), axis=-1)
    sel_o = jnp.sum(mask_o.astype(jnp.int32), axis=-1)

    def kth(sel):
        return jnp.maximum(
            1,
            jnp.round(jnp.float32(0.2) * sel.astype(jnp.float32)).astype(
                jnp.int32))

    k_h = kth(sel_h)
    k_o = kth(sel_o)
    bits_h = lax.bitcast_convert_type(basis_h, jnp.int32)
    bits_o = lax.bitcast_convert_type(basis_o, jnp.int32)

    def step(_, carry):
        lo_h, hi_h, lo_o, hi_o = carry

        def halve(lo, hi, mask, bits, k):
            mid = lo + lax.div(hi - lo, 2)
            cnt = jnp.sum(jnp.where(mask & (bits <= mid[..., None]), 1, 0),
                          axis=-1)
            ge = cnt >= k
            return jnp.where(ge, lo, mid + 1), jnp.where(ge, mid, hi)

        lo_h, hi_h = halve(lo_h, hi_h, mask_h, bits_h, k_h)
        lo_o, hi_o = halve(lo_o, hi_o, mask_o, bits_o, k_o)
        return lo_h, hi_h, lo_o, hi_o

    z = jnp.zeros((8,), jnp.int32)
    inf = jnp.full((8,), 0x7F800000, jnp.int32)
    lo_h, _, lo_o, _ = lax.fori_loop(0, 31, step, (z, inf, z, inf))

    def branch(lo, sel, mask, basis, diff_ref):
        t = lax.bitcast_convert_type(lo, jnp.float32)
        t = jnp.where(sel > 0, t, jnp.float32(1.0))[..., None]
        w = jnp.maximum((t - basis) / (t + EPS), 0.0)
        w2 = w * w
        w4 = jnp.where(mask, w2 * w2, 0.0)
        return (jnp.sum(w4[:, None, :] * diff_ref[...])
                / (jnp.sum(w4) + EPS))

    l_h = branch(lo_h, sel_h, mask_h, basis_h, diffh_ref)
    l_o = branch(lo_o, sel_o, mask_o, basis_o, diffo_ref)
    out_ref[...] = jnp.broadcast_to(l_h + l_o, (1, 1))


@functools.lru_cache(maxsize=1)
def _build_gather():
    mesh = plsc.VectorSubcoreMesh(core_axis_name="c", subcore_axis_name="s")
    return functools.partial(
        pl.kernel,
        mesh=mesh,
        out_type=[jax.ShapeDtypeStruct((BT, 1, VH), jnp.float32),
                  jax.ShapeDtypeStruct((BT, 1, VO), jnp.float32)],
        scratch_types=_GATHER_SCRATCH,
        compiler_params=pltpu.CompilerParams(needs_layout_passes=False),
    )(_gather_body)


_GATHER_SCRATCH = [
        pltpu.VMEM((VH,), jnp.float32), pltpu.VMEM((VH,), jnp.float32),
        pltpu.VMEM((VH,), jnp.float32), pltpu.VMEM((VH,), jnp.float32),
        pltpu.VMEM((VH,), jnp.float32), pltpu.VMEM((VH,), jnp.float32),
        pltpu.VMEM((VO,), jnp.float32), pltpu.VMEM((VO,), jnp.float32),
        pltpu.VMEM((VO,), jnp.float32), pltpu.VMEM((VO,), jnp.float32),
        pltpu.VMEM((VO,), jnp.float32), pltpu.VMEM((VO,), jnp.float32),
        pltpu.VMEM((_HS,), jnp.int32), pltpu.VMEM((_OS,), jnp.int32),
        pltpu.VMEM((_HS,), jnp.float32), pltpu.VMEM((_OS,), jnp.float32),
        pltpu.SemaphoreType.DMA,
]


def _gather_body(allh, allo, idxo, idxh, diffh_out, diffo_out,
                 t_ghx, t_ghy, t_ghz, t_phx, t_phy, t_phz,
                 t_gox, t_goy, t_goz, t_pox, t_poy, t_poz,
                 t_idxo, t_idxh, t_dh, t_do, sem):
    wid = lax.axis_index("s") * 2 + lax.axis_index("c")
    b = wid // _NSLICE
    s = wid % _NSLICE
    copies = [
        pltpu.async_copy(allh.at[0, b, 0, 0, :], t_ghx, sem),
        pltpu.async_copy(allh.at[0, b, 1, 0, :], t_ghy, sem),
        pltpu.async_copy(allh.at[0, b, 2, 0, :], t_ghz, sem),
        pltpu.async_copy(allh.at[1, b, 0, 0, :], t_phx, sem),
        pltpu.async_copy(allh.at[1, b, 1, 0, :], t_phy, sem),
        pltpu.async_copy(allh.at[1, b, 2, 0, :], t_phz, sem),
        pltpu.async_copy(allo.at[0, b, 0, 0, :], t_gox, sem),
        pltpu.async_copy(allo.at[0, b, 1, 0, :], t_goy, sem),
        pltpu.async_copy(allo.at[0, b, 2, 0, :], t_goz, sem),
        pltpu.async_copy(allo.at[1, b, 0, 0, :], t_pox, sem),
        pltpu.async_copy(allo.at[1, b, 1, 0, :], t_poy, sem),
        pltpu.async_copy(allo.at[1, b, 2, 0, :], t_poz, sem),
    ]

    @pl.when(s < 3)
    def _hside():
        hoff = s * _HS
        pltpu.sync_copy(idxo.at[b, 0, pl.ds(hoff, _HS)], t_idxo)
        for c in copies:
            c.wait()

        def hstep(c, carry):
            for u in range(2):
                base = c * 32 + u * 16
                idx = t_idxo[pl.ds(base, 16)]
                gx = plsc.load_gather(t_gox, [idx])
                gy = plsc.load_gather(t_goy, [idx])
                gz = plsc.load_gather(t_goz, [idx])
                px = plsc.load_gather(t_pox, [idx])
                py = plsc.load_gather(t_poy, [idx])
                pz = plsc.load_gather(t_poz, [idx])
                o = hoff + base
                dx = (px - t_phx[pl.ds(o, 16)]) - (gx - t_ghx[pl.ds(o, 16)])
                dy = (py - t_phy[pl.ds(o, 16)]) - (gy - t_ghy[pl.ds(o, 16)])
                dz = (pz - t_phz[pl.ds(o, 16)]) - (gz - t_ghz[pl.ds(o, 16)])
                t_dh[pl.ds(base, 16)] = (jnp.abs(dx) + jnp.abs(dy)
                                         + jnp.abs(dz))
            return carry

        lax.fori_loop(0, _HS // 32, hstep, 0)
        pltpu.sync_copy(t_dh, diffh_out.at[b, 0, pl.ds(hoff, _HS)])

    @pl.when(s == 3)
    def _oside():
        pltpu.sync_copy(idxh.at[b, 0, :], t_idxh)
        for c in copies:
            c.wait()

        def ostep(c, carry):
            for u in range(2):
                base = c * 32 + u * 16
                idx = t_idxh[pl.ds(base, 16)]
                gx = plsc.load_gather(t_ghx, [idx])
                gy = plsc.load_gather(t_ghy, [idx])
                gz = plsc.load_gather(t_ghz, [idx])
                px = plsc.load_gather(t_phx, [idx])
                py = plsc.load_gather(t_phy, [idx])
                pz = plsc.load_gather(t_phz, [idx])
                dx = ((px - t_pox[pl.ds(base, 16)])
                      - (gx - t_gox[pl.ds(base, 16)]))
                dy = ((py - t_poy[pl.ds(base, 16)])
                      - (gy - t_goy[pl.ds(base, 16)]))
                dz = ((pz - t_poz[pl.ds(base, 16)])
                      - (gz - t_goz[pl.ds(base, 16)]))
                t_do[pl.ds(base, 16)] = (jnp.abs(dx) + jnp.abs(dy)
                                         + jnp.abs(dz))
            return carry

        lax.fori_loop(0, _OS // 32, ostep, 0)
        pltpu.sync_copy(t_do, diffo_out.at[b, 0, :])


def _dist_call(ght, bt):
    return pl.pallas_call(
        _dist_body,
        grid=(NT,),
        in_specs=[
            pl.BlockSpec((BT, 8, TH), lambda i: (0, 0, i)),
            pl.BlockSpec((BT, 8, VO), lambda i: (0, 0, 0)),
        ],
        out_specs=[
            pl.BlockSpec((1, BT, TH), lambda i: (0, 0, i)),
            pl.BlockSpec((BT, 1, TH), lambda i: (0, 0, i)),
            pl.BlockSpec((1, BT, VO), lambda i: (0, 0, 0)),
            pl.BlockSpec((BT, 1, VO), lambda i: (0, 0, 0)),
        ],
        out_shape=[
            jax.ShapeDtypeStruct((1, BT, VH), jnp.float32),
            jax.ShapeDtypeStruct((BT, 1, VH), jnp.int32),
            jax.ShapeDtypeStruct((1, BT, VO), jnp.float32),
            jax.ShapeDtypeStruct((BT, 1, VO), jnp.int32),
        ],
        compiler_params=pltpu.CompilerParams(
            dimension_semantics=("arbitrary",),
            fuse_transposed_lhs_in_matmul=True),
    )(ght, bt)


def _loss_call(d2h, d2o, diffh, diffo):
    return pl.pallas_call(
        _loss_body,
        out_shape=jax.ShapeDtypeStruct((1, 1), jnp.float32),
    )(d2h, d2o, diffh, diffo)


def _run_gather(allh, allo, idxo, idxh):
    return _build_gather()(allh, allo, idxo, idxh)


def kernel(pred_h_verts, pred_o_verts, gt_h_verts, gt_o_verts):
    pad = VH - VH_RAW
    gh = jnp.pad(gt_h_verts, ((0, 0), (0, pad), (0, 0)), constant_values=BIGC)
    ph = jnp.pad(pred_h_verts, ((0, 0), (0, pad), (0, 0)))
    go = gt_o_verts
    po = pred_o_verts
    gh_t = jnp.transpose(gh, (0, 2, 1))
    ph_t = jnp.transpose(ph, (0, 2, 1))
    go_t = jnp.transpose(go, (0, 2, 1))
    po_t = jnp.transpose(po, (0, 2, 1))
    a2 = jnp.sum(gh * gh, axis=-1)[:, None, :]
    b2 = jnp.sum(go * go, axis=-1)[:, None, :]
    ght = jnp.concatenate(
        [gh_t, a2, jnp.ones_like(a2), jnp.zeros_like(gh_t)], axis=1)
    bt = jnp.concatenate(
        [-2.0 * go_t, jnp.ones_like(b2), b2, jnp.zeros_like(go_t)], axis=1)
    allh = jnp.stack([gh_t, ph_t], 0)[:, :, :, None, :]
    allo = jnp.stack([go_t, po_t], 0)[:, :, :, None, :]

    rmin2, idxo, cmin2, idxh = _dist_call(ght, bt)
    diffh, diffo = _run_gather(allh, allo, idxo, idxh)
    out = _loss_call(rmin2, cmin2, diffh, diffo)
    return out[0, 0]


# speedup vs baseline: 3.5889x; 1.0931x over previous
import functools

import jax
import jax.numpy as jnp
from jax import lax
from jax.experimental import pallas as pl
from jax.experimental.pallas import tpu as pltpu
from jax.experimental.pallas import tpu_sc as plsc

BT = 8
VH_RAW = 6890
VO = 2048
TH = 768
NT = 9
VH = TH * NT
THR = 0.2
EPS = 1e-8
BIGC = 1e9

_NSLICE = 4
_HS = VH // 3
_OS = VO


def _dist_body(a_ref, bt_ref, rmin_ref, rarg_ref, cmin_ref, carg_ref):
    i = pl.program_id(0)
    nch = VO // 128
    nst = TH // 8
    lane_f = lax.broadcasted_iota(
        jnp.int32, (1, 128), 1).astype(jnp.float32)
    sub_f = lax.broadcasted_iota(jnp.int32, (8, 1), 0).astype(jnp.float32)
    rmins, rargs, cmins, cargs = [], [], [], []
    for bb in range(BT):
        a = a_ref[bb]
        bt = bt_ref[bb]
        d2 = lax.dot_general(a, bt, (((0,), (0,)), ((), ())),
                             preferred_element_type=jnp.float32)
        run_min = d2[:, 0:128]
        run_ci = jnp.zeros((TH, 128), jnp.float32)
        for c in range(1, nch):
            ch = d2[:, c * 128:(c + 1) * 128]
            lt = ch < run_min
            run_min = jnp.minimum(run_min, ch)
            run_ci = jnp.where(lt, jnp.float32(c), run_ci)
        rmin = jnp.min(run_min, axis=1)
        enc = run_ci * 128.0 + jnp.broadcast_to(lane_f, (TH, 128))
        cand = jnp.where(run_min == rmin[:, None], enc, jnp.float32(4096.0))
        rmins.append(rmin)
        rargs.append(jnp.min(cand, axis=1).astype(jnp.int32))
        run_cmin = d2[0:8, :]
        run_si = jnp.zeros((8, VO), jnp.float32)
        for s_ in range(1, nst):
            st = d2[s_ * 8:(s_ + 1) * 8, :]
            lt = st < run_cmin
            run_cmin = jnp.minimum(run_cmin, st)
            run_si = jnp.where(lt, jnp.float32(s_), run_si)
        cmin_t = jnp.min(run_cmin, axis=0)
        rowenc = (run_si * 8.0 + jnp.broadcast_to(sub_f, (8, VO))
                  + (i * TH).astype(jnp.float32))
        ccand = jnp.where(run_cmin == cmin_t[None, :], rowenc,
                          jnp.float32(16384.0))
        cmins.append(cmin_t)
        cargs.append(jnp.min(ccand, axis=0).astype(jnp.int32))

    rmin_ref[0] = jnp.stack(rmins)
    rarg_ref[:, 0, :] = jnp.stack(rargs)
    cmin_all = jnp.stack(cmins)
    carg_all = jnp.stack(cargs)

    @pl.when(i == 0)
    def _init():
        cmin_ref[0] = cmin_all
        carg_ref[:, 0, :] = carg_all

    @pl.when(i != 0)
    def _acc():
        prev = cmin_ref[0]
        parg = carg_ref[:, 0, :]
        better = cmin_all < prev
        cmin_ref[0] = jnp.where(better, cmin_all, prev)
        carg_ref[:, 0, :] = jnp.where(better, carg_all, parg)


def _loss_body(d2h_ref, d2o_ref, diffh_ref, diffo_ref, out_ref):
    basis_h = jnp.sqrt(jnp.maximum(d2h_ref[0], 1e-12))
    basis_o = jnp.sqrt(jnp.maximum(d2o_ref[0], 1e-12))
    mask_h = basis_h < THR
    mask_o = basis_o < THR
    sel_h = jnp.sum(mask_h.astype(jnp.int32), axis=-1)
    sel_o = jnp.sum(mask_o.astype(jnp.int32), axis=-1)

    def kth(sel):
        return jnp.maximum(
            1,
            jnp.round(jnp.float32(0.2) * sel.astype(jnp.float32)).astype(
                jnp.int32))

    k_h = kth(sel_h)
    k_o = kth(sel_o)
    bits_h = lax.bitcast_convert_type(basis_h, jnp.int32)
    bits_o = lax.bitcast_convert_type(basis_o, jnp.int32)

    def step(_, carry):
        lo_h, hi_h, lo_o, hi_o = carry

        def halve(lo, hi, mask, bits, k):
            mid = lo + lax.div(hi - lo, 2)
            cnt = jnp.sum(jnp.where(mask & (bits <= mid[..., None]), 1, 0),
                          axis=-1)
            ge = cnt >= k
            return jnp.where(ge, lo, mid + 1), jnp.where(ge, mid, hi)

        lo_h, hi_h = halve(lo_h, hi_h, mask_h, bits_h, k_h)
        lo_o, hi_o = halve(lo_o, hi_o, mask_o, bits_o, k_o)
        return lo_h, hi_h, lo_o, hi_o

    z = jnp.zeros((8,), jnp.int32)
    inf = jnp.full((8,), 0x7F800000, jnp.int32)
    lo_h, _, lo_o, _ = lax.fori_loop(0, 31, step, (z, inf, z, inf))

    def branch(lo, sel, mask, basis, diff_ref):
        t = lax.bitcast_convert_type(lo, jnp.float32)
        t = jnp.where(sel > 0, t, jnp.float32(1.0))[..., None]
        w = jnp.maximum((t - basis) / (t + EPS), 0.0)
        w2 = w * w
        w4 = jnp.where(mask, w2 * w2, 0.0)
        return (jnp.sum(w4[:, None, :] * diff_ref[...])
                / (jnp.sum(w4) + EPS))

    l_h = branch(lo_h, sel_h, mask_h, basis_h, diffh_ref)
    l_o = branch(lo_o, sel_o, mask_o, basis_o, diffo_ref)
    out_ref[...] = jnp.broadcast_to(l_h + l_o, (1, 1))


@functools.lru_cache(maxsize=1)
def _build_gather():
    mesh = plsc.VectorSubcoreMesh(core_axis_name="c", subcore_axis_name="s")
    return functools.partial(
        pl.kernel,
        mesh=mesh,
        out_type=[jax.ShapeDtypeStruct((BT, 1, VH), jnp.float32),
                  jax.ShapeDtypeStruct((BT, 1, VO), jnp.float32)],
        scratch_types=_GATHER_SCRATCH,
        compiler_params=pltpu.CompilerParams(needs_layout_passes=False),
    )(_gather_body)


_GATHER_SCRATCH = [
        pltpu.VMEM((VH,), jnp.float32), pltpu.VMEM((VH,), jnp.float32),
        pltpu.VMEM((VH,), jnp.float32), pltpu.VMEM((VH,), jnp.float32),
        pltpu.VMEM((VH,), jnp.float32), pltpu.VMEM((VH,), jnp.float32),
        pltpu.VMEM((VO,), jnp.float32), pltpu.VMEM((VO,), jnp.float32),
        pltpu.VMEM((VO,), jnp.float32), pltpu.VMEM((VO,), jnp.float32),
        pltpu.VMEM((VO,), jnp.float32), pltpu.VMEM((VO,), jnp.float32),
        pltpu.VMEM((_HS,), jnp.int32), pltpu.VMEM((_OS,), jnp.int32),
        pltpu.VMEM((_HS,), jnp.float32), pltpu.VMEM((_OS,), jnp.float32),
        pltpu.SemaphoreType.DMA,
]


def _gather_body(allh, allo, idxo, idxh, diffh_out, diffo_out,
                 t_ghx, t_ghy, t_ghz, t_phx, t_phy, t_phz,
                 t_gox, t_goy, t_goz, t_pox, t_poy, t_poz,
                 t_idxo, t_idxh, t_dh, t_do, sem):
    wid = lax.axis_index("s") * 2 + lax.axis_index("c")
    b = wid // _NSLICE
    s = wid % _NSLICE
    copies = [
        pltpu.async_copy(allh.at[0, b, 0, 0, :], t_ghx, sem),
        pltpu.async_copy(allh.at[0, b, 1, 0, :], t_ghy, sem),
        pltpu.async_copy(allh.at[0, b, 2, 0, :], t_ghz, sem),
        pltpu.async_copy(allh.at[1, b, 0, 0, :], t_phx, sem),
        pltpu.async_copy(allh.at[1, b, 1, 0, :], t_phy, sem),
        pltpu.async_copy(allh.at[1, b, 2, 0, :], t_phz, sem),
        pltpu.async_copy(allo.at[0, b, 0, 0, :], t_gox, sem),
        pltpu.async_copy(allo.at[0, b, 1, 0, :], t_goy, sem),
        pltpu.async_copy(allo.at[0, b, 2, 0, :], t_goz, sem),
        pltpu.async_copy(allo.at[1, b, 0, 0, :], t_pox, sem),
        pltpu.async_copy(allo.at[1, b, 1, 0, :], t_poy, sem),
        pltpu.async_copy(allo.at[1, b, 2, 0, :], t_poz, sem),
    ]

    @pl.when(s < 3)
    def _hside():
        hoff = s * _HS
        pltpu.sync_copy(idxo.at[b, 0, pl.ds(hoff, _HS)], t_idxo)
        for c in copies:
            c.wait()

        def hstep(c, carry):
            for u in range(2):
                base = c * 32 + u * 16
                idx = t_idxo[pl.ds(base, 16)]
                gx = plsc.load_gather(t_gox, [idx])
                gy = plsc.load_gather(t_goy, [idx])
                gz = plsc.load_gather(t_goz, [idx])
                px = plsc.load_gather(t_pox, [idx])
                py = plsc.load_gather(t_poy, [idx])
                pz = plsc.load_gather(t_poz, [idx])
                o = hoff + base
                dx = (px - t_phx[pl.ds(o, 16)]) - (gx - t_ghx[pl.ds(o, 16)])
                dy = (py - t_phy[pl.ds(o, 16)]) - (gy - t_ghy[pl.ds(o, 16)])
                dz = (pz - t_phz[pl.ds(o, 16)]) - (gz - t_ghz[pl.ds(o, 16)])
                t_dh[pl.ds(base, 16)] = (jnp.abs(dx) + jnp.abs(dy)
                                         + jnp.abs(dz))
            return carry

        lax.fori_loop(0, _HS // 32, hstep, 0)
        pltpu.sync_copy(t_dh, diffh_out.at[b, 0, pl.ds(hoff, _HS)])

    @pl.when(s == 3)
    def _oside():
        pltpu.sync_copy(idxh.at[b, 0, :], t_idxh)
        for c in copies:
            c.wait()

        def ostep(c, carry):
            for u in range(2):
                base = c * 32 + u * 16
                idx = t_idxh[pl.ds(base, 16)]
                gx = plsc.load_gather(t_ghx, [idx])
                gy = plsc.load_gather(t_ghy, [idx])
                gz = plsc.load_gather(t_ghz, [idx])
                px = plsc.load_gather(t_phx, [idx])
                py = plsc.load_gather(t_phy, [idx])
                pz = plsc.load_gather(t_phz, [idx])
                dx = ((px - t_pox[pl.ds(base, 16)])
                      - (gx - t_gox[pl.ds(base, 16)]))
                dy = ((py - t_poy[pl.ds(base, 16)])
                      - (gy - t_goy[pl.ds(base, 16)]))
                dz = ((pz - t_poz[pl.ds(base, 16)])
                      - (gz - t_goz[pl.ds(base, 16)]))
                t_do[pl.ds(base, 16)] = (jnp.abs(dx) + jnp.abs(dy)
                                         + jnp.abs(dz))
            return carry

        lax.fori_loop(0, _OS // 32, ostep, 0)
        pltpu.sync_copy(t_do, diffo_out.at[b, 0, :])


def _dist_call(ght, bt):
    return pl.pallas_call(
        _dist_body,
        grid=(NT,),
        in_specs=[
            pl.BlockSpec((BT, 8, TH), lambda i: (0, 0, i)),
            pl.BlockSpec((BT, 8, VO), lambda i: (0, 0, 0)),
        ],
        out_specs=[
            pl.BlockSpec((1, BT, TH), lambda i: (0, 0, i)),
            pl.BlockSpec((BT, 1, TH), lambda i: (0, 0, i)),
            pl.BlockSpec((1, BT, VO), lambda i: (0, 0, 0)),
            pl.BlockSpec((BT, 1, VO), lambda i: (0, 0, 0)),
        ],
        out_shape=[
            jax.ShapeDtypeStruct((1, BT, VH), jnp.float32),
            jax.ShapeDtypeStruct((BT, 1, VH), jnp.int32),
            jax.ShapeDtypeStruct((1, BT, VO), jnp.float32),
            jax.ShapeDtypeStruct((BT, 1, VO), jnp.int32),
        ],
        compiler_params=pltpu.CompilerParams(
            dimension_semantics=("arbitrary",),
            fuse_transposed_lhs_in_matmul=True),
    )(ght, bt)


def _loss_call(d2h, d2o, diffh, diffo):
    return pl.pallas_call(
        _loss_body,
        out_shape=jax.ShapeDtypeStruct((1, 1), jnp.float32),
    )(d2h, d2o, diffh, diffo)


def _run_gather(allh, allo, idxo, idxh):
    return _build_gather()(allh, allo, idxo, idxh)


def kernel(pred_h_verts, pred_o_verts, gt_h_verts, gt_o_verts):
    pad = VH - VH_RAW
    gh = jnp.pad(gt_h_verts, ((0, 0), (0, pad), (0, 0)), constant_values=BIGC)
    ph = jnp.pad(pred_h_verts, ((0, 0), (0, pad), (0, 0)))
    go = gt_o_verts
    po = pred_o_verts
    gh_t = jnp.transpose(gh, (0, 2, 1))
    ph_t = jnp.transpose(ph, (0, 2, 1))
    go_t = jnp.transpose(go, (0, 2, 1))
    po_t = jnp.transpose(po, (0, 2, 1))
    a2 = jnp.sum(gh * gh, axis=-1)[:, None, :]
    b2 = jnp.sum(go * go, axis=-1)[:, None, :]
    ght = jnp.concatenate(
        [gh_t, a2, jnp.ones_like(a2), jnp.zeros_like(gh_t)], axis=1)
    bt = jnp.concatenate(
        [-2.0 * go_t, jnp.ones_like(b2), b2, jnp.zeros_like(go_t)], axis=1)
    allh = jnp.stack([gh_t, ph_t], 0)[:, :, :, None, :]
    allo = jnp.stack([go_t, po_t], 0)[:, :, :, None, :]

    rmin2, idxo, cmin2, idxh = _dist_call(ght, bt)
    diffh, diffo = _run_gather(allh, allo, idxo, idxh)
    out = _loss_call(rmin2, cmin2, diffh, diffo)
    return out[0, 0]


# speedup vs baseline: 3.6598x; 1.0197x over previous
import functools

import jax
import jax.numpy as jnp
from jax import lax
from jax.experimental import pallas as pl
from jax.experimental.pallas import tpu as pltpu
from jax.experimental.pallas import tpu_sc as plsc

BT = 8
VH_RAW = 6890
VO = 2048
TH = 1152
NT = 6
VH = TH * NT
THR = 0.2
EPS = 1e-8
BIGC = 1e9

_NSLICE = 4
_HS = VH // 3
_OS = VO


def _dist_body(a_ref, bt_ref, rmin_ref, rarg_ref, cmin_ref, carg_ref):
    i = pl.program_id(0)
    nch = VO // 128
    nst = TH // 8
    lane_f = lax.broadcasted_iota(
        jnp.int32, (1, 128), 1).astype(jnp.float32)
    sub_f = lax.broadcasted_iota(jnp.int32, (8, 1), 0).astype(jnp.float32)
    rmins, rargs, cmins, cargs = [], [], [], []
    for bb in range(BT):
        a = a_ref[bb]
        bt = bt_ref[bb]
        d2 = lax.dot_general(a, bt, (((0,), (0,)), ((), ())),
                             preferred_element_type=jnp.float32)
        run_min = d2[:, 0:128]
        run_ci = jnp.zeros((TH, 128), jnp.float32)
        for c in range(1, nch):
            ch = d2[:, c * 128:(c + 1) * 128]
            lt = ch < run_min
            run_min = jnp.minimum(run_min, ch)
            run_ci = jnp.where(lt, jnp.float32(c), run_ci)
        rmin = jnp.min(run_min, axis=1)
        enc = run_ci * 128.0 + jnp.broadcast_to(lane_f, (TH, 128))
        cand = jnp.where(run_min == rmin[:, None], enc, jnp.float32(4096.0))
        rmins.append(rmin)
        rargs.append(jnp.min(cand, axis=1).astype(jnp.int32))
        run_cmin = d2[0:8, :]
        run_si = jnp.zeros((8, VO), jnp.float32)
        for s_ in range(1, nst):
            st = d2[s_ * 8:(s_ + 1) * 8, :]
            lt = st < run_cmin
            run_cmin = jnp.minimum(run_cmin, st)
            run_si = jnp.where(lt, jnp.float32(s_), run_si)
        cmin_t = jnp.min(run_cmin, axis=0)
        rowenc = (run_si * 8.0 + jnp.broadcast_to(sub_f, (8, VO))
                  + (i * TH).astype(jnp.float32))
        ccand = jnp.where(run_cmin == cmin_t[None, :], rowenc,
                          jnp.float32(16384.0))
        cmins.append(cmin_t)
        cargs.append(jnp.min(ccand, axis=0).astype(jnp.int32))

    rmin_ref[0] = jnp.stack(rmins)
    rarg_ref[:, 0, :] = jnp.stack(rargs)
    cmin_all = jnp.stack(cmins)
    carg_all = jnp.stack(cargs)

    @pl.when(i == 0)
    def _init():
        cmin_ref[0] = cmin_all
        carg_ref[:, 0, :] = carg_all

    @pl.when(i != 0)
    def _acc():
        prev = cmin_ref[0]
        parg = carg_ref[:, 0, :]
        better = cmin_all < prev
        cmin_ref[0] = jnp.where(better, cmin_all, prev)
        carg_ref[:, 0, :] = jnp.where(better, carg_all, parg)


def _loss_body(d2h_ref, d2o_ref, diffh_ref, diffo_ref, out_ref):
    basis_h = jnp.sqrt(jnp.maximum(d2h_ref[0], 1e-12))
    basis_o = jnp.sqrt(jnp.maximum(d2o_ref[0], 1e-12))
    mask_h = basis_h < THR
    mask_o = basis_o < THR
    sel_h = jnp.sum(mask_h.astype(jnp.int32), axis=-1)
    sel_o = jnp.sum(mask_o.astype(jnp.int32), axis=-1)

    def kth(sel):
        return jnp.maximum(
            1,
            jnp.round(jnp.float32(0.2) * sel.astype(jnp.float32)).astype(
                jnp.int32))

    k_h = kth(sel_h)
    k_o = kth(sel_o)
    bits_h = lax.bitcast_convert_type(basis_h, jnp.int32)
    bits_o = lax.bitcast_convert_type(basis_o, jnp.int32)

    def step(_, carry):
        lo_h, hi_h, lo_o, hi_o = carry

        def halve(lo, hi, mask, bits, k):
            mid = lo + lax.div(hi - lo, 2)
            cnt = jnp.sum(jnp.where(mask & (bits <= mid[..., None]), 1, 0),
                          axis=-1)
            ge = cnt >= k
            return jnp.where(ge, lo, mid + 1), jnp.where(ge, mid, hi)

        lo_h, hi_h = halve(lo_h, hi_h, mask_h, bits_h, k_h)
        lo_o, hi_o = halve(lo_o, hi_o, mask_o, bits_o, k_o)
        return lo_h, hi_h, lo_o, hi_o

    z = jnp.zeros((8,), jnp.int32)
    inf = jnp.full((8,), 0x7F800000, jnp.int32)
    lo_h, _, lo_o, _ = lax.fori_loop(0, 31, step, (z, inf, z, inf))

    def branch(lo, sel, mask, basis, diff_ref):
        t = lax.bitcast_convert_type(lo, jnp.float32)
        t = jnp.where(sel > 0, t, jnp.float32(1.0))[..., None]
        w = jnp.maximum((t - basis) / (t + EPS), 0.0)
        w2 = w * w
        w4 = jnp.where(mask, w2 * w2, 0.0)
        return (jnp.sum(w4[:, None, :] * diff_ref[...])
                / (jnp.sum(w4) + EPS))

    l_h = branch(lo_h, sel_h, mask_h, basis_h, diffh_ref)
    l_o = branch(lo_o, sel_o, mask_o, basis_o, diffo_ref)
    out_ref[...] = jnp.broadcast_to(l_h + l_o, (1, 1))


@functools.lru_cache(maxsize=1)
def _build_gather():
    mesh = plsc.VectorSubcoreMesh(core_axis_name="c", subcore_axis_name="s")
    return functools.partial(
        pl.kernel,
        mesh=mesh,
        out_type=[jax.ShapeDtypeStruct((BT, 1, VH), jnp.float32),
                  jax.ShapeDtypeStruct((BT, 1, VO), jnp.float32)],
        scratch_types=_GATHER_SCRATCH,
        compiler_params=pltpu.CompilerParams(needs_layout_passes=False),
    )(_gather_body)


_GATHER_SCRATCH = [
        pltpu.VMEM((VH,), jnp.float32), pltpu.VMEM((VH,), jnp.float32),
        pltpu.VMEM((VH,), jnp.float32), pltpu.VMEM((VH,), jnp.float32),
        pltpu.VMEM((VH,), jnp.float32), pltpu.VMEM((VH,), jnp.float32),
        pltpu.VMEM((VO,), jnp.float32), pltpu.VMEM((VO,), jnp.float32),
        pltpu.VMEM((VO,), jnp.float32), pltpu.VMEM((VO,), jnp.float32),
        pltpu.VMEM((VO,), jnp.float32), pltpu.VMEM((VO,), jnp.float32),
        pltpu.VMEM((_HS,), jnp.int32), pltpu.VMEM((_OS,), jnp.int32),
        pltpu.VMEM((_HS,), jnp.float32), pltpu.VMEM((_OS,), jnp.float32),
        pltpu.SemaphoreType.DMA,
]


def _gather_body(allh, allo, idxo, idxh, diffh_out, diffo_out,
                 t_ghx, t_ghy, t_ghz, t_phx, t_phy, t_phz,
                 t_gox, t_goy, t_goz, t_pox, t_poy, t_poz,
                 t_idxo, t_idxh, t_dh, t_do, sem):
    wid = lax.axis_index("s") * 2 + lax.axis_index("c")
    b = wid // _NSLICE
    s = wid % _NSLICE
    copies = [
        pltpu.async_copy(allh.at[0, b, 0, 0, :], t_ghx, sem),
        pltpu.async_copy(allh.at[0, b, 1, 0, :], t_ghy, sem),
        pltpu.async_copy(allh.at[0, b, 2, 0, :], t_ghz, sem),
        pltpu.async_copy(allh.at[1, b, 0, 0, :], t_phx, sem),
        pltpu.async_copy(allh.at[1, b, 1, 0, :], t_phy, sem),
        pltpu.async_copy(allh.at[1, b, 2, 0, :], t_phz, sem),
        pltpu.async_copy(allo.at[0, b, 0, 0, :], t_gox, sem),
        pltpu.async_copy(allo.at[0, b, 1, 0, :], t_goy, sem),
        pltpu.async_copy(allo.at[0, b, 2, 0, :], t_goz, sem),
        pltpu.async_copy(allo.at[1, b, 0, 0, :], t_pox, sem),
        pltpu.async_copy(allo.at[1, b, 1, 0, :], t_poy, sem),
        pltpu.async_copy(allo.at[1, b, 2, 0, :], t_poz, sem),
    ]

    @pl.when(s < 3)
    def _hside():
        hoff = s * _HS
        pltpu.sync_copy(idxo.at[b, 0, pl.ds(hoff, _HS)], t_idxo)
        for c in copies:
            c.wait()

        def hstep(c, carry):
            for u in range(2):
                base = c * 32 + u * 16
                idx = t_idxo[pl.ds(base, 16)]
                gx = plsc.load_gather(t_gox, [idx])
                gy = plsc.load_gather(t_goy, [idx])
                gz = plsc.load_gather(t_goz, [idx])
                px = plsc.load_gather(t_pox, [idx])
                py = plsc.load_gather(t_poy, [idx])
                pz = plsc.load_gather(t_poz, [idx])
                o = hoff + base
                dx = (px - t_phx[pl.ds(o, 16)]) - (gx - t_ghx[pl.ds(o, 16)])
                dy = (py - t_phy[pl.ds(o, 16)]) - (gy - t_ghy[pl.ds(o, 16)])
                dz = (pz - t_phz[pl.ds(o, 16)]) - (gz - t_ghz[pl.ds(o, 16)])
                t_dh[pl.ds(base, 16)] = (jnp.abs(dx) + jnp.abs(dy)
                                         + jnp.abs(dz))
            return carry

        lax.fori_loop(0, _HS // 32, hstep, 0)
        pltpu.sync_copy(t_dh, diffh_out.at[b, 0, pl.ds(hoff, _HS)])

    @pl.when(s == 3)
    def _oside():
        pltpu.sync_copy(idxh.at[b, 0, :], t_idxh)
        for c in copies:
            c.wait()

        def ostep(c, carry):
            for u in range(2):
                base = c * 32 + u * 16
                idx = t_idxh[pl.ds(base, 16)]
                gx = plsc.load_gather(t_ghx, [idx])
                gy = plsc.load_gather(t_ghy, [idx])
                gz = plsc.load_gather(t_ghz, [idx])
                px = plsc.load_gather(t_phx, [idx])
                py = plsc.load_gather(t_phy, [idx])
                pz = plsc.load_gather(t_phz, [idx])
                dx = ((px - t_pox[pl.ds(base, 16)])
                      - (gx - t_gox[pl.ds(base, 16)]))
                dy = ((py - t_poy[pl.ds(base, 16)])
                      - (gy - t_goy[pl.ds(base, 16)]))
                dz = ((pz - t_poz[pl.ds(base, 16)])
                      - (gz - t_goz[pl.ds(base, 16)]))
                t_do[pl.ds(base, 16)] = (jnp.abs(dx) + jnp.abs(dy)
                                         + jnp.abs(dz))
            return carry

        lax.fori_loop(0, _OS // 32, ostep, 0)
        pltpu.sync_copy(t_do, diffo_out.at[b, 0, :])


def _dist_call(ght, bt):
    return pl.pallas_call(
        _dist_body,
        grid=(NT,),
        in_specs=[
            pl.BlockSpec((BT, 8, TH), lambda i: (0, 0, i)),
            pl.BlockSpec((BT, 8, VO), lambda i: (0, 0, 0)),
        ],
        out_specs=[
            pl.BlockSpec((1, BT, TH), lambda i: (0, 0, i)),
            pl.BlockSpec((BT, 1, TH), lambda i: (0, 0, i)),
            pl.BlockSpec((1, BT, VO), lambda i: (0, 0, 0)),
            pl.BlockSpec((BT, 1, VO), lambda i: (0, 0, 0)),
        ],
        out_shape=[
            jax.ShapeDtypeStruct((1, BT, VH), jnp.float32),
            jax.ShapeDtypeStruct((BT, 1, VH), jnp.int32),
            jax.ShapeDtypeStruct((1, BT, VO), jnp.float32),
            jax.ShapeDtypeStruct((BT, 1, VO), jnp.int32),
        ],
        compiler_params=pltpu.CompilerParams(
            dimension_semantics=("arbitrary",),
            fuse_transposed_lhs_in_matmul=True),
    )(ght, bt)


def _loss_call(d2h, d2o, diffh, diffo):
    return pl.pallas_call(
        _loss_body,
        out_shape=jax.ShapeDtypeStruct((1, 1), jnp.float32),
    )(d2h, d2o, diffh, diffo)


def _run_gather(allh, allo, idxo, idxh):
    return _build_gather()(allh, allo, idxo, idxh)


def kernel(pred_h_verts, pred_o_verts, gt_h_verts, gt_o_verts):
    pad = VH - VH_RAW
    gh = jnp.pad(gt_h_verts, ((0, 0), (0, pad), (0, 0)), constant_values=BIGC)
    ph = jnp.pad(pred_h_verts, ((0, 0), (0, pad), (0, 0)))
    go = gt_o_verts
    po = pred_o_verts
    gh_t = jnp.transpose(gh, (0, 2, 1))
    ph_t = jnp.transpose(ph, (0, 2, 1))
    go_t = jnp.transpose(go, (0, 2, 1))
    po_t = jnp.transpose(po, (0, 2, 1))
    a2 = jnp.sum(gh * gh, axis=-1)[:, None, :]
    b2 = jnp.sum(go * go, axis=-1)[:, None, :]
    ght = jnp.concatenate(
        [gh_t, a2, jnp.ones_like(a2), jnp.zeros_like(gh_t)], axis=1)
    bt = jnp.concatenate(
        [-2.0 * go_t, jnp.ones_like(b2), b2, jnp.zeros_like(go_t)], axis=1)
    allh = jnp.stack([gh_t, ph_t], 0)[:, :, :, None, :]
    allo = jnp.stack([go_t, po_t], 0)[:, :, :, None, :]

    rmin2, idxo, cmin2, idxh = _dist_call(ght, bt)
    diffh, diffo = _run_gather(allh, allo, idxo, idxh)
    out = _loss_call(rmin2, cmin2, diffh, diffo)
    return out[0, 0]
